# baseline (speedup 1.0000x reference)
import functools

import jax
import jax.numpy as jnp
from jax import lax
from jax.experimental import pallas as pl
from jax.experimental.pallas import tpu as pltpu

_N = 10000
_E = 320000
_T = 16
_G = 512
_EMB = 32
_H = 128
_D = 128
_OUT = 25

_PREC = lax.Precision.HIGHEST


def _leaky(v):
    return jnp.where(v > 0, v, 0.1 * v)


def _dot(a, b):
    return jnp.dot(a, b, precision=_PREC, preferred_element_type=jnp.float32)


def _encode_body(xr_ref, wblk_ref, bemb_ref, wih_ref, whh_ref, bih_ref,
                 bhh_ref, wdyn_ref, bdyn_ref, wtd_ref, wts_ref,
                 hist_ref, td_ref, ts_ref):
    emb = _leaky(_dot(xr_ref[:], wblk_ref[:]) + bemb_ref[:])
    R = emb.shape[0]
    h = jnp.zeros((R, _H), jnp.float32)
    for t in range(_T):
        e_t = emb[:, t * _EMB:(t + 1) * _EMB]
        gi = _dot(e_t, wih_ref[:]) + bih_ref[:]
        gh = _dot(h, whh_ref[:]) + bhh_ref[:]
        r = jax.nn.sigmoid(gi[:, :_H] + gh[:, :_H])
        z = jax.nn.sigmoid(gi[:, _H:2 * _H] + gh[:, _H:2 * _H])
        n = jnp.tanh(gi[:, 2 * _H:] + r * gh[:, 2 * _H:])
        h = (1.0 - z) * n + z * h
    hist = _leaky(_dot(_leaky(h), wdyn_ref[:]) + bdyn_ref[:])
    hist_ref[:] = hist
    td_ref[:] = _dot(hist, wtd_ref[:])
    ts_ref[:] = _dot(hist, wts_ref[:])


def _encode(xr, wblk, bemb, wih, whh, bih, bhh, wdyn, bdyn, wtd, wts):
    R = 2000
    grid = _N // R
    full = lambda s: pl.BlockSpec(s, lambda i: (0, 0))
    return pl.pallas_call(
        _encode_body,
        grid=(grid,),
        in_specs=[
            pl.BlockSpec((R, 2 * _T), lambda i: (i, 0)),
            full((2 * _T, _T * _EMB)),
            full((1, _T * _EMB)),
            full((_EMB, 3 * _H)),
            full((_H, 3 * _H)),
            full((1, 3 * _H)),
            full((1, 3 * _H)),
            full((_H, _H)),
            full((1, _H)),
            full((_H, 2 * _H)),
            full((_H, 2 * _H)),
        ],
        out_specs=[
            pl.BlockSpec((R, _H), lambda i: (i, 0)),
            pl.BlockSpec((R, 2 * _H), lambda i: (i, 0)),
            pl.BlockSpec((R, 2 * _H), lambda i: (i, 0)),
        ],
        out_shape=[
            jax.ShapeDtypeStruct((_N, _H), jnp.float32),
            jax.ShapeDtypeStruct((_N, 2 * _H), jnp.float32),
            jax.ShapeDtypeStruct((_N, 2 * _H), jnp.float32),
        ],
    )(xr, wblk, bemb, wih, whh, bih, bhh, wdyn, bdyn, wtd, wts)


def _edge_body(u_ref, v_ref, ea_ref, wfe_ref, wse_ref, bf_ref, bs_ref, m_ref):
    u = u_ref[:]
    v = v_ref[:]
    ea0 = ea_ref[:, 0:1]
    ea1 = ea_ref[:, 1:2]
    cf = ea0 * wfe_ref[0:1, :] + ea1 * wfe_ref[1:2, :] + bf_ref[:]
    cs = ea0 * wse_ref[0:1, :] + ea1 * wse_ref[1:2, :] + bs_ref[:]
    gf = u[:, :_H] + v[:, :_H] + cf
    gs = u[:, _H:] + v[:, _H:] + cs
    sp = jnp.maximum(gs, 0.0) + jnp.log1p(jnp.exp(-jnp.abs(gs)))
    m_ref[:] = jax.nn.sigmoid(gf) * sp


def _edge_m(u, v, ea, wfe, wse, bf, bs):
    R = 8000
    grid = _E // R
    full = lambda s: pl.BlockSpec(s, lambda i: (0, 0))
    return pl.pallas_call(
        _edge_body,
        grid=(grid,),
        in_specs=[
            pl.BlockSpec((R, 2 * _H), lambda i: (i, 0)),
            pl.BlockSpec((R, 2 * _H), lambda i: (i, 0)),
            pl.BlockSpec((R, 2), lambda i: (i, 0)),
            full((2, _H)),
            full((2, _H)),
            full((1, _H)),
            full((1, _H)),
        ],
        out_specs=pl.BlockSpec((R, _H), lambda i: (i, 0)),
        out_shape=jax.ShapeDtypeStruct((_E, _H), jnp.float32),
    )(u, v, ea, wfe, wse, bf, bs)


def _stats_body(agg_ref, s_ref, ss_ref):
    i = pl.program_id(0)

    @pl.when(i == 0)
    def _():
        s_ref[:] = jnp.zeros_like(s_ref)
        ss_ref[:] = jnp.zeros_like(ss_ref)

    a = agg_ref[:]
    s_ref[:] += jnp.sum(a, axis=0, keepdims=True)
    ss_ref[:] += jnp.sum(a * a, axis=0, keepdims=True)


def _stats(agg):
    R = 1000
    return pl.pallas_call(
        _stats_body,
        grid=(_N // R,),
        in_specs=[pl.BlockSpec((R, _H), lambda i: (i, 0))],
        out_specs=[pl.BlockSpec((1, _H), lambda i: (0, 0)),
                   pl.BlockSpec((1, _H), lambda i: (0, 0))],
        out_shape=[jax.ShapeDtypeStruct((1, _H), jnp.float32),
                   jax.ShapeDtypeStruct((1, _H), jnp.float32)],
    )(agg)


def _norm_scale(s, ss, g_ref):
    mu = s * (1.0 / _N)
    var = ss * (1.0 / _N) - mu * mu
    return mu, g_ref[:] * lax.rsqrt(var + 1e-5)


def _bn_tables_body(agg_ref, xn_ref, s_ref, ss_ref, g_ref, b_ref, wtd_ref,
                    wts_ref, f_ref, td_ref, ts_ref):
    mu, scale = _norm_scale(s_ref[:], ss_ref[:], g_ref)
    fn = xn_ref[:] + (agg_ref[:] - mu) * scale + b_ref[:]
    f_ref[:] = fn
    td_ref[:] = _dot(fn, wtd_ref[:])
    ts_ref[:] = _dot(fn, wts_ref[:])


def _bn_tables(agg, xn, gamma, beta, wtd, wts):
    s, ss = _stats(agg)
    R = 2000
    full = lambda s_: pl.BlockSpec(s_, lambda i: (0, 0))
    row = lambda w: pl.BlockSpec((R, w), lambda i: (i, 0))
    return pl.pallas_call(
        _bn_tables_body,
        grid=(_N // R,),
        in_specs=[
            row(_H), row(_H), full((1, _H)), full((1, _H)), full((1, _H)),
            full((1, _H)), full((_H, 2 * _H)), full((_H, 2 * _H)),
        ],
        out_specs=[row(_H), row(2 * _H), row(2 * _H)],
        out_shape=[
            jax.ShapeDtypeStruct((_N, _H), jnp.float32),
            jax.ShapeDtypeStruct((_N, 2 * _H), jnp.float32),
            jax.ShapeDtypeStruct((_N, 2 * _H), jnp.float32),
        ],
    )(agg, xn, s, ss, gamma, beta, wtd, wts)


def _bn_tgt_body(agg_ref, xn_ref, s_ref, ss_ref, g_ref, b_ref, batch_ref,
                 f_ref, tgt_ref):
    i = pl.program_id(0)
    mu, scale = _norm_scale(s_ref[:], ss_ref[:], g_ref)
    f_ref[:] = xn_ref[:] + (agg_ref[:] - mu) * scale + b_ref[:]

    @pl.when(i == 0)
    def _():
        tgt_ref[:] = jnp.zeros_like(tgt_ref)

    gids = lax.broadcasted_iota(jnp.int32, (1, _G), 1)
    b = batch_ref[:]
    tgt_ref[:] += jnp.sum((b < gids).astype(jnp.int32), axis=0, keepdims=True)


def _bn_tgt(agg, xn, gamma, beta, batch_col):
    s, ss = _stats(agg)
    R = 2000
    full = lambda s_: pl.BlockSpec(s_, lambda i: (0, 0))
    row = lambda w: pl.BlockSpec((R, w), lambda i: (i, 0))
    return pl.pallas_call(
        _bn_tgt_body,
        grid=(_N // R,),
        in_specs=[
            row(_H), row(_H), full((1, _H)), full((1, _H)), full((1, _H)),
            full((1, _H)), row(1),
        ],
        out_specs=[row(_H), pl.BlockSpec((1, _G), lambda i: (0, 0))],
        out_shape=[
            jax.ShapeDtypeStruct((_N, _H), jnp.float32),
            jax.ShapeDtypeStruct((1, _G), jnp.int32),
        ],
    )(agg, xn, s, ss, gamma, beta, batch_col)


def _decode_body(ht_ref, ft_ref, wnbr_ref, bnbr_ref, w1ih_ref, w1hh_ref,
                 b1_ref, w2ih_ref, w2hh_ref, b2_ref, wop_ref, bop_ref,
                 out_ref, h2all):
    tar = _leaky(_dot(ft_ref[:], wnbr_ref[:]) + bnbr_ref[:])
    gi1 = (_dot(ht_ref[:], w1ih_ref[0:_H, :])
           + _dot(tar, w1ih_ref[_H:2 * _H, :]) + b1_ref[:])
    h1 = jnp.zeros((_G, _D), jnp.float32)
    c1 = jnp.zeros((_G, _D), jnp.float32)
    h2 = jnp.zeros((_G, _D), jnp.float32)
    c2 = jnp.zeros((_G, _D), jnp.float32)
    for t in range(_OUT):
        g1 = gi1 + _dot(h1, w1hh_ref[:])
        i1 = jax.nn.sigmoid(g1[:, :_D])
        f1 = jax.nn.sigmoid(g1[:, _D:2 * _D])
        gg1 = jnp.tanh(g1[:, 2 * _D:3 * _D])
        o1 = jax.nn.sigmoid(g1[:, 3 * _D:])
        c1 = f1 * c1 + i1 * gg1
        h1 = o1 * jnp.tanh(c1)
        g2 = _dot(h1, w2ih_ref[:]) + _dot(h2, w2hh_ref[:]) + b2_ref[:]
        i2 = jax.nn.sigmoid(g2[:, :_D])
        f2 = jax.nn.sigmoid(g2[:, _D:2 * _D])
        gg2 = jnp.tanh(g2[:, 2 * _D:3 * _D])
        o2 = jax.nn.sigmoid(g2[:, 3 * _D:])
        c2 = f2 * c2 + i2 * gg2
        h2 = o2 * jnp.tanh(c2)
        h2all[:, t * _D:(t + 1) * _D] = h2
    out_ref[:] = _dot(h2all[:], wop_ref[:]) + bop_ref[:]


def _decode(ht, ft, wnbr, bnbr, w1ih, w1hh, b1, w2ih, w2hh, b2, wop, bop):
    full = lambda s: pl.BlockSpec(s, lambda: (0, 0))
    return pl.pallas_call(
        _decode_body,
        in_specs=[
            full((_G, _H)), full((_G, _H)), full((_H, _H)), full((1, _H)),
            full((2 * _H, 4 * _D)), full((_D, 4 * _D)), full((1, 4 * _D)),
            full((_D, 4 * _D)), full((_D, 4 * _D)), full((1, 4 * _D)),
            full((_OUT * _D, _OUT * 2)), full((1, _OUT * 2)),
        ],
        out_specs=full((_G, _OUT * 2)),
        out_shape=jax.ShapeDtypeStruct((_G, _OUT * 2), jnp.float32),
        scratch_shapes=[pltpu.VMEM((_G, _OUT * _D), jnp.float32)],
    )(ht, ft, wnbr, bnbr, w1ih, w1hh, b1, w2ih, w2hh, b2, wop, bop)


def kernel(x, edge_index, edge_attr, batch, num_graphs, W_ip, b_ip, gru_Wih,
           gru_Whh, gru_bih, gru_bhh, W_dyn, b_dyn, cgc1_Wf, cgc1_bf, cgc1_Ws,
           cgc1_bs, cgc1_gamma, cgc1_beta, cgc2_Wf, cgc2_bf, cgc2_Ws, cgc2_bs,
           cgc2_gamma, cgc2_beta, W_nbr, b_nbr, lstm1_Wih, lstm1_Whh,
           lstm1_bih, lstm1_bhh, lstm2_Wih, lstm2_Whh, lstm2_bih, lstm2_bhh,
           W_op, b_op):
    f32 = jnp.float32
    xr = x.reshape(_N, _T * 2)
    wblk = jax.scipy.linalg.block_diag(*([W_ip.T] * _T))
    bemb = jnp.tile(b_ip, _T)[None, :]
    hist, td1, ts1 = _encode(
        xr, wblk, bemb, gru_Wih.T, gru_Whh.T, gru_bih[None, :],
        gru_bhh[None, :], W_dyn.T, b_dyn[None, :],
        jnp.concatenate([cgc1_Wf[:, :_H].T, cgc1_Ws[:, :_H].T], axis=1),
        jnp.concatenate([cgc1_Wf[:, _H:2 * _H].T, cgc1_Ws[:, _H:2 * _H].T],
                        axis=1))

    src = edge_index[0]
    dst = edge_index[1]

    u1 = jnp.take(td1, dst, axis=0)
    v1 = jnp.take(ts1, src, axis=0)
    m1 = _edge_m(u1, v1, edge_attr, cgc1_Wf[:, 2 * _H:].T,
                 cgc1_Ws[:, 2 * _H:].T, cgc1_bf[None, :], cgc1_bs[None, :])
    agg1 = jax.ops.segment_sum(m1, dst, num_segments=_N)

    f1, td2, ts2 = _bn_tables(
        agg1, hist, cgc1_gamma[None, :], cgc1_beta[None, :],
        jnp.concatenate([cgc2_Wf[:, :_H].T, cgc2_Ws[:, :_H].T], axis=1),
        jnp.concatenate([cgc2_Wf[:, _H:2 * _H].T, cgc2_Ws[:, _H:2 * _H].T],
                        axis=1))

    u2 = jnp.take(td2, dst, axis=0)
    v2 = jnp.take(ts2, src, axis=0)
    m2 = _edge_m(u2, v2, edge_attr, cgc2_Wf[:, 2 * _H:].T,
                 cgc2_Ws[:, 2 * _H:].T, cgc2_bf[None, :], cgc2_bs[None, :])
    agg2 = jax.ops.segment_sum(m2, dst, num_segments=_N)

    f2, tgt2d = _bn_tgt(agg2, f1, cgc2_gamma[None, :], cgc2_beta[None, :],
                        batch.reshape(_N, 1))
    tgt = tgt2d.reshape(_G)

    hist_tgt = jnp.take(hist, tgt, axis=0)
    f_tgt = jnp.take(f2, tgt, axis=0)

    wop_blk = jax.scipy.linalg.block_diag(*([W_op.T] * _OUT))
    bop = jnp.tile(b_op, _OUT)[None, :]
    b1c = (lstm1_bih + lstm1_bhh)[None, :]
    b2c = (lstm2_bih + lstm2_bhh)[None, :]
    out = _decode(hist_tgt, f_tgt, W_nbr.T, b_nbr[None, :], lstm1_Wih.T,
                  lstm1_Whh.T, b1c, lstm2_Wih.T, lstm2_Whh.T, b2c,
                  wop_blk, bop)
    return out.reshape(_G, _OUT, 2).astype(f32)


# speedup vs baseline: 1.4136x; 1.4136x over previous
import functools

import jax
import jax.numpy as jnp
from jax import lax
from jax.experimental import pallas as pl
from jax.experimental.pallas import tpu as pltpu
from jax.experimental.pallas import tpu_sc as plsc

_N = 10000
_E = 320000
_T = 16
_G = 512
_EMB = 32
_H = 128
_D = 128
_OUT = 25

_NW = 32
_CH = 128
_EW = 10240
_EP = _NW * _EW
_NCH = _EW // _CH
_NP = 10240

_PREC = lax.Precision.HIGHEST


def _leaky(v):
    return jnp.where(v > 0, v, 0.1 * v)


def _dot(a, b):
    return jnp.dot(a, b, precision=_PREC, preferred_element_type=jnp.float32)


def _dotd(a, b):
    return jnp.dot(a, b, preferred_element_type=jnp.float32)


def _sc_mesh():
    return plsc.VectorSubcoreMesh(core_axis_name="c", subcore_axis_name="s",
                                  num_cores=2, num_subcores=16)


def _sc_gather_uv(table_pad, dst_pad, src_pad):
    @functools.partial(
        pl.kernel,
        out_type=[jax.ShapeDtypeStruct((_EP, _H), jnp.float32),
                  jax.ShapeDtypeStruct((_EP, _H), jnp.float32)],
        mesh=_sc_mesh(),
        scratch_types=[
            pltpu.VMEM((_CH,), jnp.int32),
            pltpu.VMEM((_CH,), jnp.int32),
            pltpu.VMEM((_CH, _H), jnp.float32),
            pltpu.VMEM((_CH, _H), jnp.float32),
            pltpu.SemaphoreType.DMA,
            pltpu.SemaphoreType.DMA,
        ],
    )
    def k(tab_hbm, dst_hbm, src_hbm, u_hbm, v_hbm, idxd, idxs, rowd, rows,
          semd, sems):
        wid = lax.axis_index("s") * 2 + lax.axis_index("c")
        base0 = pl.multiple_of(wid * _EW, _CH)

        def body(ci, carry):
            base = pl.multiple_of(base0 + ci * _CH, _CH)
            pltpu.sync_copy(dst_hbm.at[pl.ds(base, _CH)], idxd)
            pltpu.sync_copy(src_hbm.at[pl.ds(base, _CH)], idxs)
            cd = pltpu.async_copy(tab_hbm.at[idxd], rowd, semd)
            cs = pltpu.async_copy(tab_hbm.at[idxs], rows, sems)
            cd.wait()
            cs.wait()
            pltpu.sync_copy(rowd, u_hbm.at[pl.ds(base, _CH)])
            pltpu.sync_copy(rows, v_hbm.at[pl.ds(base, _CH)])
            return carry

        lax.fori_loop(0, _NCH, body, 0)

    return k(table_pad, dst_pad, src_pad)


def _sc_scatter(m, dst_pad, zrows):
    @functools.partial(
        pl.kernel,
        out_type=jax.ShapeDtypeStruct((2, _NP, _H), jnp.float32),
        mesh=_sc_mesh(),
        scratch_types=[
            pltpu.VMEM((_CH,), jnp.int32),
            pltpu.VMEM((_CH, _H), jnp.float32),
            pltpu.VMEM_SHARED((_NP, _H), jnp.float32),
        ],
    )
    def k(m_hbm, dst_hbm, z_hbm, out_hbm, idx, mbuf, acc):
        c = lax.axis_index("c")
        s = lax.axis_index("s")
        wid = s * 2 + c
        zr = _NP // 16
        pltpu.sync_copy(z_hbm.at[pl.ds(s * zr, zr)], acc.at[pl.ds(s * zr, zr)])
        plsc.subcore_barrier()

        base0 = pl.multiple_of(wid * _EW, _CH)

        def body(ci, carry):
            base = pl.multiple_of(base0 + ci * _CH, _CH)
            pltpu.sync_copy(dst_hbm.at[pl.ds(base, _CH)], idx)
            pltpu.sync_copy(m_hbm.at[pl.ds(base, _CH)], mbuf)
            pltpu.sync_copy(mbuf, acc.at[idx], add=True)
            return carry

        lax.fori_loop(0, _NCH, body, 0)
        plsc.subcore_barrier()
        pltpu.sync_copy(acc.at[pl.ds(s * zr, zr)],
                        out_hbm.at[c, pl.ds(s * zr, zr)])

    return k(m, dst_pad, zrows)


def _embed_body(x2_ref, wip_ref, bip_ref, e_ref):
    e_ref[:] = _leaky(_dotd(x2_ref[:], wip_ref[:]) + bip_ref[:])


def _embed(x2, wip, bip):
    R = 16000
    NT = _N * _T
    return pl.pallas_call(
        _embed_body,
        grid=(NT // R,),
        in_specs=[
            pl.BlockSpec((R, 2), lambda i: (i, 0)),
            pl.BlockSpec((2, _EMB), lambda i: (0, 0)),
            pl.BlockSpec((1, _EMB), lambda i: (0, 0)),
        ],
        out_specs=pl.BlockSpec((R, _EMB), lambda i: (i, 0)),
        out_shape=jax.ShapeDtypeStruct((NT, _EMB), jnp.float32),
    )(x2, wip, bip)


def _encode_body(emb_ref, wih_ref, whh_ref, bih_ref,
                 bhh_ref, wdyn_ref, bdyn_ref, hist_ref):
    R = emb_ref.shape[0]
    h = jnp.zeros((R, _H), jnp.float32)
    for t in range(_T):
        e_t = emb_ref[:, t * _EMB:(t + 1) * _EMB]
        gi = _dotd(e_t, wih_ref[:]) + bih_ref[:]
        gh = _dotd(h, whh_ref[:]) + bhh_ref[:]
        r = jax.nn.sigmoid(gi[:, :_H] + gh[:, :_H])
        z = jax.nn.sigmoid(gi[:, _H:2 * _H] + gh[:, _H:2 * _H])
        n = jnp.tanh(gi[:, 2 * _H:] + r * gh[:, 2 * _H:])
        h = (1.0 - z) * n + z * h
    hist_ref[:] = _leaky(_dotd(_leaky(h), wdyn_ref[:]) + bdyn_ref[:])


def _encode(emb, wih, whh, bih, bhh, wdyn, bdyn):
    R = 2000
    full = lambda s: pl.BlockSpec(s, lambda i: (0, 0))
    return pl.pallas_call(
        _encode_body,
        grid=(_N // R,),
        in_specs=[
            pl.BlockSpec((R, _T * _EMB), lambda i: (i, 0)),
            full((_EMB, 3 * _H)),
            full((_H, 3 * _H)),
            full((1, 3 * _H)),
            full((1, 3 * _H)),
            full((_H, _H)),
            full((1, _H)),
        ],
        out_specs=pl.BlockSpec((R, _H), lambda i: (i, 0)),
        out_shape=jax.ShapeDtypeStruct((_N, _H), jnp.float32),
    )(emb, wih, whh, bih, bhh, wdyn, bdyn)


def _edge_body(u_ref, v_ref, ea_ref, wd_ref, ws_ref, wfe_ref, wse_ref,
               bf_ref, bs_ref, m_ref):
    g = _dot(u_ref[:], wd_ref[:]) + _dot(v_ref[:], ws_ref[:])
    ea0 = ea_ref[:, 0:1]
    ea1 = ea_ref[:, 1:2]
    gf = (g[:, :_H] + ea0 * wfe_ref[0:1, :] + ea1 * wfe_ref[1:2, :]
          + bf_ref[:])
    gs = (g[:, _H:] + ea0 * wse_ref[0:1, :] + ea1 * wse_ref[1:2, :]
          + bs_ref[:])
    sp = jnp.maximum(gs, 0.0) + jnp.log1p(jnp.exp(-jnp.abs(gs)))
    m_ref[:] = jax.nn.sigmoid(gf) * sp


def _edge_m(u, v, ea, wd, ws, wfe, wse, bf, bs):
    R = 2048
    grid = _EP // R
    full = lambda s: pl.BlockSpec(s, lambda i: (0, 0))
    return pl.pallas_call(
        _edge_body,
        grid=(grid,),
        in_specs=[
            pl.BlockSpec((R, _H), lambda i: (i, 0)),
            pl.BlockSpec((R, _H), lambda i: (i, 0)),
            pl.BlockSpec((R, 2), lambda i: (i, 0)),
            full((_H, 2 * _H)),
            full((_H, 2 * _H)),
            full((2, _H)),
            full((2, _H)),
            full((1, _H)),
            full((1, _H)),
        ],
        out_specs=pl.BlockSpec((R, _H), lambda i: (i, 0)),
        out_shape=jax.ShapeDtypeStruct((_EP, _H), jnp.float32),
    )(u, v, ea, wd, ws, wfe, wse, bf, bs)


def _sum_body(a0_ref, a1_ref, s_ref):
    i = pl.program_id(0)

    @pl.when(i == 0)
    def _():
        s_ref[:] = jnp.zeros_like(s_ref)

    a = a0_ref[:] + a1_ref[:]
    s_ref[:] += jnp.sum(a, axis=0, keepdims=True)


def _var_body(a0_ref, a1_ref, s_ref, v_ref):
    i = pl.program_id(0)

    @pl.when(i == 0)
    def _():
        v_ref[:] = jnp.zeros_like(v_ref)

    d = a0_ref[:] + a1_ref[:] - s_ref[:] * (1.0 / _N)
    v_ref[:] += jnp.sum(d * d, axis=0, keepdims=True)


def _stats(a0, a1):
    R = 1000
    row = pl.BlockSpec((R, _H), lambda i: (i, 0))
    one = pl.BlockSpec((1, _H), lambda i: (0, 0))
    s = pl.pallas_call(
        _sum_body,
        grid=(_N // R,),
        in_specs=[row, row],
        out_specs=one,
        out_shape=jax.ShapeDtypeStruct((1, _H), jnp.float32),
    )(a0, a1)
    v = pl.pallas_call(
        _var_body,
        grid=(_N // R,),
        in_specs=[row, row, one],
        out_specs=one,
        out_shape=jax.ShapeDtypeStruct((1, _H), jnp.float32),
    )(a0, a1, s)
    return s, v


def _norm_scale(s, ss, g_ref):
    mu = s * (1.0 / _N)
    var = ss * (1.0 / _N)
    return mu, g_ref[:] * lax.rsqrt(var + 1e-5)


def _bn_body(a0_ref, a1_ref, xn_ref, s_ref, ss_ref, g_ref, b_ref, f_ref):
    mu, scale = _norm_scale(s_ref[:], ss_ref[:], g_ref)
    f_ref[:] = xn_ref[:] + (a0_ref[:] + a1_ref[:] - mu) * scale + b_ref[:]


def _bn(a0, a1, xn, gamma, beta):
    s, ss = _stats(a0, a1)
    R = 2000
    full = lambda s_: pl.BlockSpec(s_, lambda i: (0, 0))
    row = lambda w: pl.BlockSpec((R, w), lambda i: (i, 0))
    return pl.pallas_call(
        _bn_body,
        grid=(_N // R,),
        in_specs=[
            row(_H), row(_H), row(_H), full((1, _H)), full((1, _H)),
            full((1, _H)), full((1, _H)),
        ],
        out_specs=row(_H),
        out_shape=jax.ShapeDtypeStruct((_N, _H), jnp.float32),
    )(a0, a1, xn, s, ss, gamma, beta)


def _bn_tgt_body(a0_ref, a1_ref, xn_ref, s_ref, ss_ref, g_ref, b_ref,
                 batch_ref, f_ref, tgt_ref):
    i = pl.program_id(0)
    mu, scale = _norm_scale(s_ref[:], ss_ref[:], g_ref)
    f_ref[:] = (xn_ref[:] + (a0_ref[:] + a1_ref[:] - mu) * scale + b_ref[:])

    @pl.when(i == 0)
    def _():
        tgt_ref[:] = jnp.zeros_like(tgt_ref)

    gids = lax.broadcasted_iota(jnp.int32, (1, _G), 1)
    b = batch_ref[:]
    tgt_ref[:] += jnp.sum((b < gids).astype(jnp.int32), axis=0, keepdims=True)

    @pl.when(i == pl.num_programs(0) - 1)
    def _():
        tgt_ref[:] = jnp.minimum(tgt_ref[:], _N - 1)


def _bn_tgt(a0, a1, xn, gamma, beta, batch_col):
    s, ss = _stats(a0, a1)
    R = 2000
    full = lambda s_: pl.BlockSpec(s_, lambda i: (0, 0))
    row = lambda w: pl.BlockSpec((R, w), lambda i: (i, 0))
    return pl.pallas_call(
        _bn_tgt_body,
        grid=(_N // R,),
        in_specs=[
            row(_H), row(_H), row(_H), full((1, _H)), full((1, _H)),
            full((1, _H)), full((1, _H)), row(1),
        ],
        out_specs=[row(_H), pl.BlockSpec((1, _G), lambda i: (0, 0))],
        out_shape=[
            jax.ShapeDtypeStruct((_N, _H), jnp.float32),
            jax.ShapeDtypeStruct((1, _G), jnp.int32),
        ],
    )(a0, a1, xn, s, ss, gamma, beta, batch_col)


def _decode_body(hist_ref, f2_ref, tgt_ref, wnbr_ref, bnbr_ref, w1ih_ref,
                 w1hh_ref, b1_ref, w2ih_ref, w2hh_ref, b2_ref, wop_ref,
                 bop_ref, out_ref, h2all):
    ht = jnp.zeros((_G, _H), jnp.float32)
    ft = jnp.zeros((_G, _H), jnp.float32)
    C = 1000
    dn = (((0,), (0,)), ((), ()))
    for ci in range(_N // C):
        ni = lax.broadcasted_iota(jnp.int32, (C, 1), 0) + ci * C
        oh = (ni == tgt_ref[:]).astype(jnp.float32)
        ht += lax.dot_general(oh, hist_ref[pl.ds(ci * C, C), :], dn,
                              precision=_PREC,
                              preferred_element_type=jnp.float32)
        ft += lax.dot_general(oh, f2_ref[pl.ds(ci * C, C), :], dn,
                              precision=_PREC,
                              preferred_element_type=jnp.float32)

    tar = _leaky(_dotd(ft, wnbr_ref[:]) + bnbr_ref[:])
    enc = jnp.concatenate([ht, tar], axis=1)
    gi1 = _dotd(enc, w1ih_ref[:]) + b1_ref[:]
    h1 = jnp.zeros((_G, _D), jnp.float32)
    c1 = jnp.zeros((_G, _D), jnp.float32)
    h2 = jnp.zeros((_G, _D), jnp.float32)
    c2 = jnp.zeros((_G, _D), jnp.float32)
    for t in range(_OUT):
        g1 = gi1 + _dotd(h1, w1hh_ref[:])
        i1 = jax.nn.sigmoid(g1[:, :_D])
        f1 = jax.nn.sigmoid(g1[:, _D:2 * _D])
        gg1 = jnp.tanh(g1[:, 2 * _D:3 * _D])
        o1 = jax.nn.sigmoid(g1[:, 3 * _D:])
        c1 = f1 * c1 + i1 * gg1
        h1 = o1 * jnp.tanh(c1)
        g2 = _dotd(h1, w2ih_ref[:]) + _dotd(h2, w2hh_ref[:]) + b2_ref[:]
        i2 = jax.nn.sigmoid(g2[:, :_D])
        f2g = jax.nn.sigmoid(g2[:, _D:2 * _D])
        gg2 = jnp.tanh(g2[:, 2 * _D:3 * _D])
        o2 = jax.nn.sigmoid(g2[:, 3 * _D:])
        c2 = f2g * c2 + i2 * gg2
        h2 = o2 * jnp.tanh(c2)
        h2all[:, t * _D:(t + 1) * _D] = h2
    out_ref[:] = _dotd(h2all[:], wop_ref[:]) + bop_ref[:]


def _decode(hist, f2, tgt, wnbr, bnbr, w1ih, w1hh, b1, w2ih, w2hh, b2, wop,
            bop):
    full = lambda s: pl.BlockSpec(s, lambda: (0, 0))
    return pl.pallas_call(
        _decode_body,
        in_specs=[
            full((_N, _H)), full((_N, _H)), full((1, _G)),
            full((_H, _H)), full((1, _H)),
            full((2 * _H, 4 * _D)), full((_D, 4 * _D)), full((1, 4 * _D)),
            full((_D, 4 * _D)), full((_D, 4 * _D)), full((1, 4 * _D)),
            full((_OUT * _D, _OUT * 2)), full((1, _OUT * 2)),
        ],
        out_specs=full((_G, _OUT * 2)),
        out_shape=jax.ShapeDtypeStruct((_G, _OUT * 2), jnp.float32),
        scratch_shapes=[pltpu.VMEM((_G, _OUT * _D), jnp.float32)],
    )(hist, f2, tgt, wnbr, bnbr, w1ih, w1hh, b1, w2ih, w2hh, b2, wop, bop)


def _cgconv_layer(xn, dst_pad, src_pad, ea_pad, zrows, Wf, bf, Ws, bs):
    xn_pad = jnp.concatenate(
        [xn, jnp.zeros((_NP - _N, _H), jnp.float32)], axis=0)
    u, v = _sc_gather_uv(xn_pad, dst_pad, src_pad)
    wd = jnp.concatenate([Wf[:, :_H].T, Ws[:, :_H].T], axis=1)
    ws = jnp.concatenate([Wf[:, _H:2 * _H].T, Ws[:, _H:2 * _H].T], axis=1)
    m = _edge_m(u, v, ea_pad, wd, ws, Wf[:, 2 * _H:].T, Ws[:, 2 * _H:].T,
                bf[None, :], bs[None, :])
    slabs = _sc_scatter(m, dst_pad, zrows)
    return slabs[0], slabs[1]


def kernel(x, edge_index, edge_attr, batch, num_graphs, W_ip, b_ip, gru_Wih,
           gru_Whh, gru_bih, gru_bhh, W_dyn, b_dyn, cgc1_Wf, cgc1_bf, cgc1_Ws,
           cgc1_bs, cgc1_gamma, cgc1_beta, cgc2_Wf, cgc2_bf, cgc2_Ws, cgc2_bs,
           cgc2_gamma, cgc2_beta, W_nbr, b_nbr, lstm1_Wih, lstm1_Whh,
           lstm1_bih, lstm1_bhh, lstm2_Wih, lstm2_Whh, lstm2_bih, lstm2_bhh,
           W_op, b_op):
    emb2 = _embed(x.reshape(_N * _T, 2), W_ip.T, b_ip[None, :])
    emb = emb2.reshape(_N, _T * _EMB)
    hist = _encode(emb, gru_Wih.T, gru_Whh.T, gru_bih[None, :],
                   gru_bhh[None, :], W_dyn.T, b_dyn[None, :])

    pad_e = _EP - _E
    dst_pad = jnp.concatenate(
        [edge_index[1], jnp.full((pad_e,), _N, jnp.int32)])
    src_pad = jnp.concatenate(
        [edge_index[0], jnp.full((pad_e,), _N, jnp.int32)])
    ea_pad = jnp.concatenate(
        [edge_attr, jnp.zeros((pad_e, 2), jnp.float32)], axis=0)
    zrows = jnp.zeros((_NP, _H), jnp.float32)

    a0, a1 = _cgconv_layer(hist, dst_pad, src_pad, ea_pad, zrows,
                           cgc1_Wf, cgc1_bf, cgc1_Ws, cgc1_bs)
    f1 = _bn(a0, a1, hist, cgc1_gamma[None, :], cgc1_beta[None, :])

    b0, b1 = _cgconv_layer(f1, dst_pad, src_pad, ea_pad, zrows,
                           cgc2_Wf, cgc2_bf, cgc2_Ws, cgc2_bs)
    f2, tgt2d = _bn_tgt(b0, b1, f1, cgc2_gamma[None, :], cgc2_beta[None, :],
                        batch.reshape(_N, 1))

    wop_blk = jax.scipy.linalg.block_diag(*([W_op.T] * _OUT))
    bop = jnp.tile(b_op, _OUT)[None, :]
    b1c = (lstm1_bih + lstm1_bhh)[None, :]
    b2c = (lstm2_bih + lstm2_bhh)[None, :]
    out = _decode(hist, f2, tgt2d, W_nbr.T, b_nbr[None, :], lstm1_Wih.T,
                  lstm1_Whh.T, b1c, lstm2_Wih.T, lstm2_Whh.T, b2c,
                  wop_blk, bop)
    return out.reshape(_G, _OUT, 2)


# speedup vs baseline: 1.5723x; 1.1123x over previous
import functools

import jax
import jax.numpy as jnp
from jax import lax
from jax.experimental import pallas as pl
from jax.experimental.pallas import tpu as pltpu
from jax.experimental.pallas import tpu_sc as plsc

_N = 10000
_E = 320000
_T = 16
_G = 512
_EMB = 32
_H = 128
_D = 128
_OUT = 25

_NW = 32
_CH = 128
_EW = 10240
_EP = _NW * _EW
_NCH = _EW // _CH
_NP = 10240

_PREC = lax.Precision.HIGHEST


def _leaky(v):
    return jnp.where(v > 0, v, 0.1 * v)


def _dot(a, b):
    return jnp.dot(a, b, precision=_PREC, preferred_element_type=jnp.float32)


def _dotd(a, b):
    return jnp.dot(a, b, preferred_element_type=jnp.float32)


def _sc_mesh():
    return plsc.VectorSubcoreMesh(core_axis_name="c", subcore_axis_name="s",
                                  num_cores=2, num_subcores=16)


def _sc_gather_uv(table_pad, dst_pad, src_pad):
    @functools.partial(
        pl.kernel,
        out_type=[jax.ShapeDtypeStruct((_EP, _H), jnp.float32),
                  jax.ShapeDtypeStruct((_EP, _H), jnp.float32)],
        mesh=_sc_mesh(),
        scratch_types=[
            pltpu.VMEM((_CH,), jnp.int32),
            pltpu.VMEM((_CH,), jnp.int32),
            pltpu.VMEM((_CH,), jnp.int32),
            pltpu.VMEM((_CH,), jnp.int32),
            pltpu.VMEM((_CH, _H), jnp.float32),
            pltpu.VMEM((_CH, _H), jnp.float32),
            pltpu.VMEM((_CH, _H), jnp.float32),
            pltpu.VMEM((_CH, _H), jnp.float32),
            pltpu.SemaphoreType.DMA,
            pltpu.SemaphoreType.DMA,
            pltpu.SemaphoreType.DMA,
            pltpu.SemaphoreType.DMA,
        ],
    )
    def k(tab_hbm, dst_hbm, src_hbm, u_hbm, v_hbm, idxd0, idxs0, idxd1,
          idxs1, u0, v0, u1, v1, su0, sv0, su1, sv1):
        wid = lax.axis_index("s") * 2 + lax.axis_index("c")
        base0 = pl.multiple_of(wid * _EW, _CH)
        sets = ((idxd0, idxs0, u0, v0, su0, sv0),
                (idxd1, idxs1, u1, v1, su1, sv1))

        def start(ci, st):
            idxd, idxs, ub, vb, su, sv = st
            base = pl.multiple_of(base0 + ci * _CH, _CH)
            pltpu.sync_copy(dst_hbm.at[pl.ds(base, _CH)], idxd)
            pltpu.sync_copy(src_hbm.at[pl.ds(base, _CH)], idxs)
            pltpu.async_copy(tab_hbm.at[idxd], ub, su)
            pltpu.async_copy(tab_hbm.at[idxs], vb, sv)

        def finish(ci, st):
            idxd, idxs, ub, vb, su, sv = st
            pltpu.make_async_copy(tab_hbm.at[idxd], ub, su).wait()
            pltpu.make_async_copy(tab_hbm.at[idxs], vb, sv).wait()
            base = pl.multiple_of(base0 + ci * _CH, _CH)
            pltpu.sync_copy(ub, u_hbm.at[pl.ds(base, _CH)])
            pltpu.sync_copy(vb, v_hbm.at[pl.ds(base, _CH)])

        npair = _NCH // 2
        start(0, sets[0])

        def body(pi, carry):
            c0 = pi * 2
            start(c0 + 1, sets[1])
            finish(c0, sets[0])

            @pl.when(pi < npair - 1)
            def _():
                start(c0 + 2, sets[0])

            finish(c0 + 1, sets[1])
            return carry

        lax.fori_loop(0, npair, body, 0)

    return k(table_pad, dst_pad, src_pad)


def _sc_scatter(m, dst_pad, zrows):
    @functools.partial(
        pl.kernel,
        out_type=jax.ShapeDtypeStruct((2, _NP, _H), jnp.float32),
        mesh=_sc_mesh(),
        scratch_types=[
            pltpu.VMEM((_CH,), jnp.int32),
            pltpu.VMEM((_CH,), jnp.int32),
            pltpu.VMEM((_CH, _H), jnp.float32),
            pltpu.VMEM((_CH, _H), jnp.float32),
            pltpu.VMEM_SHARED((_NP, _H), jnp.float32),
            pltpu.SemaphoreType.DMA,
            pltpu.SemaphoreType.DMA,
        ],
    )
    def k(m_hbm, dst_hbm, z_hbm, out_hbm, idx0, idx1, m0, m1, acc, sm0, sm1):
        c = lax.axis_index("c")
        s = lax.axis_index("s")
        wid = s * 2 + c
        zr = _NP // 16
        pltpu.sync_copy(z_hbm.at[pl.ds(s * zr, zr)], acc.at[pl.ds(s * zr, zr)])
        plsc.subcore_barrier()

        base0 = pl.multiple_of(wid * _EW, _CH)
        sets = ((idx0, m0, sm0), (idx1, m1, sm1))

        def start(ci, st):
            idx, mb, sm = st
            base = pl.multiple_of(base0 + ci * _CH, _CH)
            pltpu.sync_copy(dst_hbm.at[pl.ds(base, _CH)], idx)
            pltpu.async_copy(m_hbm.at[pl.ds(base, _CH)], mb, sm)

        def finish(ci, st):
            idx, mb, sm = st
            base = pl.multiple_of(base0 + ci * _CH, _CH)
            pltpu.make_async_copy(m_hbm.at[pl.ds(base, _CH)], mb, sm).wait()
            pltpu.sync_copy(mb, acc.at[idx], add=True)

        npair = _NCH // 2
        start(0, sets[0])

        def body(pi, carry):
            c0 = pi * 2
            start(c0 + 1, sets[1])
            finish(c0, sets[0])

            @pl.when(pi < npair - 1)
            def _():
                start(c0 + 2, sets[0])

            finish(c0 + 1, sets[1])
            return carry

        lax.fori_loop(0, npair, body, 0)
        plsc.subcore_barrier()
        pltpu.sync_copy(acc.at[pl.ds(s * zr, zr)],
                        out_hbm.at[c, pl.ds(s * zr, zr)])

    return k(m, dst_pad, zrows)


def _embed_body(x2_ref, wip_ref, bip_ref, e_ref):
    e_ref[:] = _leaky(_dotd(x2_ref[:], wip_ref[:]) + bip_ref[:])


def _embed(x2, wip, bip):
    R = 16000
    NT = _N * _T
    return pl.pallas_call(
        _embed_body,
        grid=(NT // R,),
        in_specs=[
            pl.BlockSpec((R, 2), lambda i: (i, 0)),
            pl.BlockSpec((2, _EMB), lambda i: (0, 0)),
            pl.BlockSpec((1, _EMB), lambda i: (0, 0)),
        ],
        out_specs=pl.BlockSpec((R, _EMB), lambda i: (i, 0)),
        out_shape=jax.ShapeDtypeStruct((NT, _EMB), jnp.float32),
    )(x2, wip, bip)


def _encode_body(emb_ref, wih_ref, whh_ref, bih_ref,
                 bhh_ref, wdyn_ref, bdyn_ref, hist_ref):
    R = emb_ref.shape[0]
    h = jnp.zeros((R, _H), jnp.float32)
    for t in range(_T):
        e_t = emb_ref[:, t * _EMB:(t + 1) * _EMB]
        gi = _dotd(e_t, wih_ref[:]) + bih_ref[:]
        gh = _dotd(h, whh_ref[:]) + bhh_ref[:]
        r = jax.nn.sigmoid(gi[:, :_H] + gh[:, :_H])
        z = jax.nn.sigmoid(gi[:, _H:2 * _H] + gh[:, _H:2 * _H])
        n = jnp.tanh(gi[:, 2 * _H:] + r * gh[:, 2 * _H:])
        h = (1.0 - z) * n + z * h
    hist_ref[:] = _leaky(_dotd(_leaky(h), wdyn_ref[:]) + bdyn_ref[:])


def _encode(emb, wih, whh, bih, bhh, wdyn, bdyn):
    R = 2000
    full = lambda s: pl.BlockSpec(s, lambda i: (0, 0))
    return pl.pallas_call(
        _encode_body,
        grid=(_N // R,),
        in_specs=[
            pl.BlockSpec((R, _T * _EMB), lambda i: (i, 0)),
            full((_EMB, 3 * _H)),
            full((_H, 3 * _H)),
            full((1, 3 * _H)),
            full((1, 3 * _H)),
            full((_H, _H)),
            full((1, _H)),
        ],
        out_specs=pl.BlockSpec((R, _H), lambda i: (i, 0)),
        out_shape=jax.ShapeDtypeStruct((_N, _H), jnp.float32),
    )(emb, wih, whh, bih, bhh, wdyn, bdyn)


def _edge_body(u_ref, v_ref, ea_ref, wd_ref, ws_ref, wfe_ref, wse_ref,
               bf_ref, bs_ref, m_ref):
    g = _dot(u_ref[:], wd_ref[:]) + _dot(v_ref[:], ws_ref[:])
    ea0 = ea_ref[:, 0:1]
    ea1 = ea_ref[:, 1:2]
    gf = (g[:, :_H] + ea0 * wfe_ref[0:1, :] + ea1 * wfe_ref[1:2, :]
          + bf_ref[:])
    gs = (g[:, _H:] + ea0 * wse_ref[0:1, :] + ea1 * wse_ref[1:2, :]
          + bs_ref[:])
    sp = jnp.maximum(gs, 0.0) + jnp.log1p(jnp.exp(-jnp.abs(gs)))
    m_ref[:] = jax.nn.sigmoid(gf) * sp


def _edge_m(u, v, ea, wd, ws, wfe, wse, bf, bs):
    R = 2048
    grid = _EP // R
    full = lambda s: pl.BlockSpec(s, lambda i: (0, 0))
    return pl.pallas_call(
        _edge_body,
        grid=(grid,),
        in_specs=[
            pl.BlockSpec((R, _H), lambda i: (i, 0)),
            pl.BlockSpec((R, _H), lambda i: (i, 0)),
            pl.BlockSpec((R, 2), lambda i: (i, 0)),
            full((_H, 2 * _H)),
            full((_H, 2 * _H)),
            full((2, _H)),
            full((2, _H)),
            full((1, _H)),
            full((1, _H)),
        ],
        out_specs=pl.BlockSpec((R, _H), lambda i: (i, 0)),
        out_shape=jax.ShapeDtypeStruct((_EP, _H), jnp.float32),
    )(u, v, ea, wd, ws, wfe, wse, bf, bs)


def _sum_body(a0_ref, a1_ref, s_ref):
    i = pl.program_id(0)

    @pl.when(i == 0)
    def _():
        s_ref[:] = jnp.zeros_like(s_ref)

    a = a0_ref[:] + a1_ref[:]
    s_ref[:] += jnp.sum(a, axis=0, keepdims=True)


def _var_body(a0_ref, a1_ref, s_ref, v_ref):
    i = pl.program_id(0)

    @pl.when(i == 0)
    def _():
        v_ref[:] = jnp.zeros_like(v_ref)

    d = a0_ref[:] + a1_ref[:] - s_ref[:] * (1.0 / _N)
    v_ref[:] += jnp.sum(d * d, axis=0, keepdims=True)


def _stats(a0, a1):
    R = 1000
    row = pl.BlockSpec((R, _H), lambda i: (i, 0))
    one = pl.BlockSpec((1, _H), lambda i: (0, 0))
    s = pl.pallas_call(
        _sum_body,
        grid=(_N // R,),
        in_specs=[row, row],
        out_specs=one,
        out_shape=jax.ShapeDtypeStruct((1, _H), jnp.float32),
    )(a0, a1)
    v = pl.pallas_call(
        _var_body,
        grid=(_N // R,),
        in_specs=[row, row, one],
        out_specs=one,
        out_shape=jax.ShapeDtypeStruct((1, _H), jnp.float32),
    )(a0, a1, s)
    return s, v


def _norm_scale(s, ss, g_ref):
    mu = s * (1.0 / _N)
    var = ss * (1.0 / _N)
    return mu, g_ref[:] * lax.rsqrt(var + 1e-5)


def _bn_body(a0_ref, a1_ref, xn_ref, s_ref, ss_ref, g_ref, b_ref, f_ref):
    mu, scale = _norm_scale(s_ref[:], ss_ref[:], g_ref)
    f_ref[:] = xn_ref[:] + (a0_ref[:] + a1_ref[:] - mu) * scale + b_ref[:]


def _bn(a0, a1, xn, gamma, beta):
    s, ss = _stats(a0, a1)
    R = 2000
    full = lambda s_: pl.BlockSpec(s_, lambda i: (0, 0))
    row = lambda w: pl.BlockSpec((R, w), lambda i: (i, 0))
    return pl.pallas_call(
        _bn_body,
        grid=(_N // R,),
        in_specs=[
            row(_H), row(_H), row(_H), full((1, _H)), full((1, _H)),
            full((1, _H)), full((1, _H)),
        ],
        out_specs=row(_H),
        out_shape=jax.ShapeDtypeStruct((_N, _H), jnp.float32),
    )(a0, a1, xn, s, ss, gamma, beta)


def _bn_tgt_body(a0_ref, a1_ref, xn_ref, s_ref, ss_ref, g_ref, b_ref,
                 batch_ref, f_ref, tgt_ref):
    i = pl.program_id(0)
    mu, scale = _norm_scale(s_ref[:], ss_ref[:], g_ref)
    f_ref[:] = (xn_ref[:] + (a0_ref[:] + a1_ref[:] - mu) * scale + b_ref[:])

    @pl.when(i == 0)
    def _():
        tgt_ref[:] = jnp.zeros_like(tgt_ref)

    gids = lax.broadcasted_iota(jnp.int32, (1, _G), 1)
    b = batch_ref[:]
    tgt_ref[:] += jnp.sum((b < gids).astype(jnp.int32), axis=0, keepdims=True)

    @pl.when(i == pl.num_programs(0) - 1)
    def _():
        tgt_ref[:] = jnp.minimum(tgt_ref[:], _N - 1)


def _bn_tgt(a0, a1, xn, gamma, beta, batch_col):
    s, ss = _stats(a0, a1)
    R = 2000
    full = lambda s_: pl.BlockSpec(s_, lambda i: (0, 0))
    row = lambda w: pl.BlockSpec((R, w), lambda i: (i, 0))
    return pl.pallas_call(
        _bn_tgt_body,
        grid=(_N // R,),
        in_specs=[
            row(_H), row(_H), row(_H), full((1, _H)), full((1, _H)),
            full((1, _H)), full((1, _H)), row(1),
        ],
        out_specs=[row(_H), pl.BlockSpec((1, _G), lambda i: (0, 0))],
        out_shape=[
            jax.ShapeDtypeStruct((_N, _H), jnp.float32),
            jax.ShapeDtypeStruct((1, _G), jnp.int32),
        ],
    )(a0, a1, xn, s, ss, gamma, beta, batch_col)


def _decode_body(hist_ref, f2_ref, tgt_ref, wnbr_ref, bnbr_ref, w1ih_ref,
                 w1hh_ref, b1_ref, w2ih_ref, w2hh_ref, b2_ref, wop_ref,
                 bop_ref, out_ref, h2all):
    ht = jnp.zeros((_G, _H), jnp.float32)
    ft = jnp.zeros((_G, _H), jnp.float32)
    C = 1000
    dn = (((0,), (0,)), ((), ()))
    for ci in range(_N // C):
        ni = lax.broadcasted_iota(jnp.int32, (C, 1), 0) + ci * C
        oh = (ni == tgt_ref[:]).astype(jnp.float32)
        ht += lax.dot_general(oh, hist_ref[pl.ds(ci * C, C), :], dn,
                              precision=_PREC,
                              preferred_element_type=jnp.float32)
        ft += lax.dot_general(oh, f2_ref[pl.ds(ci * C, C), :], dn,
                              precision=_PREC,
                              preferred_element_type=jnp.float32)

    tar = _leaky(_dotd(ft, wnbr_ref[:]) + bnbr_ref[:])
    enc = jnp.concatenate([ht, tar], axis=1)
    gi1 = _dotd(enc, w1ih_ref[:]) + b1_ref[:]
    h1 = jnp.zeros((_G, _D), jnp.float32)
    c1 = jnp.zeros((_G, _D), jnp.float32)
    h2 = jnp.zeros((_G, _D), jnp.float32)
    c2 = jnp.zeros((_G, _D), jnp.float32)
    for t in range(_OUT):
        g1 = gi1 + _dotd(h1, w1hh_ref[:])
        i1 = jax.nn.sigmoid(g1[:, :_D])
        f1 = jax.nn.sigmoid(g1[:, _D:2 * _D])
        gg1 = jnp.tanh(g1[:, 2 * _D:3 * _D])
        o1 = jax.nn.sigmoid(g1[:, 3 * _D:])
        c1 = f1 * c1 + i1 * gg1
        h1 = o1 * jnp.tanh(c1)
        g2 = _dotd(h1, w2ih_ref[:]) + _dotd(h2, w2hh_ref[:]) + b2_ref[:]
        i2 = jax.nn.sigmoid(g2[:, :_D])
        f2g = jax.nn.sigmoid(g2[:, _D:2 * _D])
        gg2 = jnp.tanh(g2[:, 2 * _D:3 * _D])
        o2 = jax.nn.sigmoid(g2[:, 3 * _D:])
        c2 = f2g * c2 + i2 * gg2
        h2 = o2 * jnp.tanh(c2)
        h2all[:, t * _D:(t + 1) * _D] = h2
    out_ref[:] = _dotd(h2all[:], wop_ref[:]) + bop_ref[:]


def _decode(hist, f2, tgt, wnbr, bnbr, w1ih, w1hh, b1, w2ih, w2hh, b2, wop,
            bop):
    full = lambda s: pl.BlockSpec(s, lambda: (0, 0))
    return pl.pallas_call(
        _decode_body,
        in_specs=[
            full((_N, _H)), full((_N, _H)), full((1, _G)),
            full((_H, _H)), full((1, _H)),
            full((2 * _H, 4 * _D)), full((_D, 4 * _D)), full((1, 4 * _D)),
            full((_D, 4 * _D)), full((_D, 4 * _D)), full((1, 4 * _D)),
            full((_OUT * _D, _OUT * 2)), full((1, _OUT * 2)),
        ],
        out_specs=full((_G, _OUT * 2)),
        out_shape=jax.ShapeDtypeStruct((_G, _OUT * 2), jnp.float32),
        scratch_shapes=[pltpu.VMEM((_G, _OUT * _D), jnp.float32)],
    )(hist, f2, tgt, wnbr, bnbr, w1ih, w1hh, b1, w2ih, w2hh, b2, wop, bop)


def _cgconv_layer(xn, dst_pad, src_pad, ea_pad, zrows, Wf, bf, Ws, bs):
    xn_pad = jnp.concatenate(
        [xn, jnp.zeros((_NP - _N, _H), jnp.float32)], axis=0)
    u, v = _sc_gather_uv(xn_pad, dst_pad, src_pad)
    wd = jnp.concatenate([Wf[:, :_H].T, Ws[:, :_H].T], axis=1)
    ws = jnp.concatenate([Wf[:, _H:2 * _H].T, Ws[:, _H:2 * _H].T], axis=1)
    m = _edge_m(u, v, ea_pad, wd, ws, Wf[:, 2 * _H:].T, Ws[:, 2 * _H:].T,
                bf[None, :], bs[None, :])
    slabs = _sc_scatter(m, dst_pad, zrows)
    return slabs[0], slabs[1]


def kernel(x, edge_index, edge_attr, batch, num_graphs, W_ip, b_ip, gru_Wih,
           gru_Whh, gru_bih, gru_bhh, W_dyn, b_dyn, cgc1_Wf, cgc1_bf, cgc1_Ws,
           cgc1_bs, cgc1_gamma, cgc1_beta, cgc2_Wf, cgc2_bf, cgc2_Ws, cgc2_bs,
           cgc2_gamma, cgc2_beta, W_nbr, b_nbr, lstm1_Wih, lstm1_Whh,
           lstm1_bih, lstm1_bhh, lstm2_Wih, lstm2_Whh, lstm2_bih, lstm2_bhh,
           W_op, b_op):
    emb2 = _embed(x.reshape(_N * _T, 2), W_ip.T, b_ip[None, :])
    emb = emb2.reshape(_N, _T * _EMB)
    hist = _encode(emb, gru_Wih.T, gru_Whh.T, gru_bih[None, :],
                   gru_bhh[None, :], W_dyn.T, b_dyn[None, :])

    pad_e = _EP - _E
    dst_pad = jnp.concatenate(
        [edge_index[1], jnp.full((pad_e,), _N, jnp.int32)])
    src_pad = jnp.concatenate(
        [edge_index[0], jnp.full((pad_e,), _N, jnp.int32)])
    ea_pad = jnp.concatenate(
        [edge_attr, jnp.zeros((pad_e, 2), jnp.float32)], axis=0)
    zrows = jnp.zeros((_NP, _H), jnp.float32)

    a0, a1 = _cgconv_layer(hist, dst_pad, src_pad, ea_pad, zrows,
                           cgc1_Wf, cgc1_bf, cgc1_Ws, cgc1_bs)
    f1 = _bn(a0, a1, hist, cgc1_gamma[None, :], cgc1_beta[None, :])

    b0, b1 = _cgconv_layer(f1, dst_pad, src_pad, ea_pad, zrows,
                           cgc2_Wf, cgc2_bf, cgc2_Ws, cgc2_bs)
    f2, tgt2d = _bn_tgt(b0, b1, f1, cgc2_gamma[None, :], cgc2_beta[None, :],
                        batch.reshape(_N, 1))

    wop_blk = jax.scipy.linalg.block_diag(*([W_op.T] * _OUT))
    bop = jnp.tile(b_op, _OUT)[None, :]
    b1c = (lstm1_bih + lstm1_bhh)[None, :]
    b2c = (lstm2_bih + lstm2_bhh)[None, :]
    out = _decode(hist, f2, tgt2d, W_nbr.T, b_nbr[None, :], lstm1_Wih.T,
                  lstm1_Whh.T, b1c, lstm2_Wih.T, lstm2_Whh.T, b2c,
                  wop_blk, bop)
    return out.reshape(_G, _OUT, 2)


# speedup vs baseline: 2.0377x; 1.2960x over previous
import functools

import jax
import jax.numpy as jnp
from jax import lax
from jax.experimental import pallas as pl
from jax.experimental.pallas import tpu as pltpu
from jax.experimental.pallas import tpu_sc as plsc

_N = 10000
_E = 320000
_T = 16
_G = 512
_EMB = 32
_H = 128
_D = 128
_OUT = 25

_NW = 32
_CH = 128
_EW = 10240
_EP = _NW * _EW
_NCH = _EW // _CH
_NP = 10240

_PREC = lax.Precision.HIGHEST


def _leaky(v):
    return jnp.where(v > 0, v, 0.1 * v)


def _dot(a, b):
    return jnp.dot(a, b, precision=_PREC, preferred_element_type=jnp.float32)


def _dotd(a, b):
    return jnp.dot(a, b, preferred_element_type=jnp.float32)


def _sc_mesh():
    return plsc.VectorSubcoreMesh(core_axis_name="c", subcore_axis_name="s",
                                  num_cores=2, num_subcores=16)


def _sc_gather_uv(table_pad, dst2d, src2d):
    @functools.partial(
        pl.kernel,
        out_type=[jax.ShapeDtypeStruct((_EP, _H), jnp.float32),
                  jax.ShapeDtypeStruct((_EP, _H), jnp.float32)],
        mesh=_sc_mesh(),
        scratch_types=[
            pltpu.VMEM((_NCH, _CH), jnp.int32),
            pltpu.VMEM((_NCH, _CH), jnp.int32),
            pltpu.VMEM((_CH, _H), jnp.float32),
            pltpu.VMEM((_CH, _H), jnp.float32),
            pltpu.VMEM((_CH, _H), jnp.float32),
            pltpu.VMEM((_CH, _H), jnp.float32),
            pltpu.SemaphoreType.DMA,
            pltpu.SemaphoreType.DMA,
            pltpu.SemaphoreType.DMA,
            pltpu.SemaphoreType.DMA,
            pltpu.SemaphoreType.DMA,
            pltpu.SemaphoreType.DMA,
            pltpu.SemaphoreType.DMA,
            pltpu.SemaphoreType.DMA,
        ],
    )
    def k(tab_hbm, dst_hbm, src_hbm, u_hbm, v_hbm, idxd, idxs, u0, v0, u1,
          v1, su0, sv0, su1, sv1, wu0, wv0, wu1, wv1):
        wid = lax.axis_index("s") * 2 + lax.axis_index("c")
        base0 = pl.multiple_of(wid * _EW, _CH)
        pltpu.sync_copy(dst_hbm.at[pl.ds(wid * _NCH, _NCH)], idxd)
        pltpu.sync_copy(src_hbm.at[pl.ds(wid * _NCH, _NCH)], idxs)
        sets = ((u0, v0, su0, sv0, wu0, wv0),
                (u1, v1, su1, sv1, wu1, wv1))

        def startg(ci, st):
            ub, vb, su, sv, wu, wv = st
            pltpu.async_copy(tab_hbm.at[idxd.at[ci]], ub, su)
            pltpu.async_copy(tab_hbm.at[idxs.at[ci]], vb, sv)

        def finishg(ci, st):
            ub, vb, su, sv, wu, wv = st
            pltpu.make_async_copy(tab_hbm.at[idxd.at[ci]], ub, su).wait()
            pltpu.make_async_copy(tab_hbm.at[idxs.at[ci]], vb, sv).wait()
            base = pl.multiple_of(base0 + ci * _CH, _CH)
            pltpu.async_copy(ub, u_hbm.at[pl.ds(base, _CH)], wu)
            pltpu.async_copy(vb, v_hbm.at[pl.ds(base, _CH)], wv)

        def waitw(st):
            ub, vb, su, sv, wu, wv = st
            pltpu.make_async_copy(ub, u_hbm.at[pl.ds(0, _CH)], wu).wait()
            pltpu.make_async_copy(vb, v_hbm.at[pl.ds(0, _CH)], wv).wait()

        npair = _NCH // 2
        startg(0, sets[0])

        def body(pi, carry):
            c0 = pi * 2

            @pl.when(pi > 0)
            def _():
                waitw(sets[1])

            startg(c0 + 1, sets[1])
            finishg(c0, sets[0])

            @pl.when(pi < npair - 1)
            def _():
                waitw(sets[0])
                startg(c0 + 2, sets[0])

            finishg(c0 + 1, sets[1])
            return carry

        lax.fori_loop(0, npair, body, 0)
        waitw(sets[0])
        waitw(sets[1])

    return k(table_pad, dst2d, src2d)


def _sc_scatter(m, dst_pad, zrows):
    @functools.partial(
        pl.kernel,
        out_type=jax.ShapeDtypeStruct((2, _NP, _H), jnp.float32),
        mesh=_sc_mesh(),
        scratch_types=[
            pltpu.VMEM((_NCH, _CH), jnp.int32),
            pltpu.VMEM((_CH, _H), jnp.float32),
            pltpu.VMEM((_CH, _H), jnp.float32),
            pltpu.VMEM_SHARED((_NP, _H), jnp.float32),
            pltpu.SemaphoreType.DMA,
            pltpu.SemaphoreType.DMA,
        ],
    )
    def k(m_hbm, dst_hbm, z_hbm, out_hbm, idx, m0, m1, acc, sm0, sm1):
        c = lax.axis_index("c")
        s = lax.axis_index("s")
        wid = s * 2 + c
        zr = _NP // 16
        pltpu.sync_copy(z_hbm.at[pl.ds(s * zr, zr)], acc.at[pl.ds(s * zr, zr)])
        pltpu.sync_copy(dst_hbm.at[pl.ds(wid * _NCH, _NCH)], idx)
        plsc.subcore_barrier()

        base0 = pl.multiple_of(wid * _EW, _CH)
        sets = ((m0, sm0), (m1, sm1))

        def start(ci, st):
            mb, sm = st
            base = pl.multiple_of(base0 + ci * _CH, _CH)
            pltpu.async_copy(m_hbm.at[pl.ds(base, _CH)], mb, sm)

        def finish(ci, st):
            mb, sm = st
            base = pl.multiple_of(base0 + ci * _CH, _CH)
            pltpu.make_async_copy(m_hbm.at[pl.ds(base, _CH)], mb, sm).wait()
            pltpu.sync_copy(mb, acc.at[idx.at[ci]], add=True)

        npair = _NCH // 2
        start(0, sets[0])

        def body(pi, carry):
            c0 = pi * 2
            start(c0 + 1, sets[1])
            finish(c0, sets[0])

            @pl.when(pi < npair - 1)
            def _():
                start(c0 + 2, sets[0])

            finish(c0 + 1, sets[1])
            return carry

        lax.fori_loop(0, npair, body, 0)
        plsc.subcore_barrier()
        pltpu.sync_copy(acc.at[pl.ds(s * zr, zr)],
                        out_hbm.at[c, pl.ds(s * zr, zr)])

    return k(m, dst_pad, zrows)


def _embed_body(x2_ref, wip_ref, bip_ref, e_ref):
    e_ref[:] = _leaky(_dotd(x2_ref[:], wip_ref[:]) + bip_ref[:])


def _embed(x2, wip, bip):
    R = 16000
    NT = _N * _T
    return pl.pallas_call(
        _embed_body,
        grid=(NT // R,),
        in_specs=[
            pl.BlockSpec((R, 2), lambda i: (i, 0)),
            pl.BlockSpec((2, _EMB), lambda i: (0, 0)),
            pl.BlockSpec((1, _EMB), lambda i: (0, 0)),
        ],
        out_specs=pl.BlockSpec((R, _EMB), lambda i: (i, 0)),
        out_shape=jax.ShapeDtypeStruct((NT, _EMB), jnp.float32),
    )(x2, wip, bip)


def _encode_body(emb_ref, wih_ref, whh_ref, bih_ref,
                 bhh_ref, wdyn_ref, bdyn_ref, hist_ref):
    R = emb_ref.shape[0]
    h = jnp.zeros((R, _H), jnp.float32)
    for t in range(_T):
        e_t = emb_ref[:, t * _EMB:(t + 1) * _EMB]
        gi = _dotd(e_t, wih_ref[:]) + bih_ref[:]
        gh = _dotd(h, whh_ref[:]) + bhh_ref[:]
        r = jax.nn.sigmoid(gi[:, :_H] + gh[:, :_H])
        z = jax.nn.sigmoid(gi[:, _H:2 * _H] + gh[:, _H:2 * _H])
        n = jnp.tanh(gi[:, 2 * _H:] + r * gh[:, 2 * _H:])
        h = (1.0 - z) * n + z * h
    hist_ref[:] = _leaky(_dotd(_leaky(h), wdyn_ref[:]) + bdyn_ref[:])


def _encode(emb, wih, whh, bih, bhh, wdyn, bdyn):
    R = 2000
    full = lambda s: pl.BlockSpec(s, lambda i: (0, 0))
    return pl.pallas_call(
        _encode_body,
        grid=(_N // R,),
        in_specs=[
            pl.BlockSpec((R, _T * _EMB), lambda i: (i, 0)),
            full((_EMB, 3 * _H)),
            full((_H, 3 * _H)),
            full((1, 3 * _H)),
            full((1, 3 * _H)),
            full((_H, _H)),
            full((1, _H)),
        ],
        out_specs=pl.BlockSpec((R, _H), lambda i: (i, 0)),
        out_shape=jax.ShapeDtypeStruct((_N, _H), jnp.float32),
    )(emb, wih, whh, bih, bhh, wdyn, bdyn)


def _edge_body(u_ref, v_ref, ea_ref, wf_ref, ws_ref, bf_ref, bs_ref, m_ref):
    z = jnp.concatenate([u_ref[:], v_ref[:], ea_ref[:]], axis=1)
    gf = _dotd(z, wf_ref[:]) + bf_ref[:]
    gs = _dotd(z, ws_ref[:]) + bs_ref[:]
    sp = jnp.maximum(gs, 0.0) + jnp.log1p(jnp.exp(-jnp.abs(gs)))
    m_ref[:] = jax.nn.sigmoid(gf) * sp


def _edge_m(u, v, ea, wf, ws, bf, bs):
    R = 2048
    grid = _EP // R
    full = lambda s: pl.BlockSpec(s, lambda i: (0, 0))
    return pl.pallas_call(
        _edge_body,
        grid=(grid,),
        in_specs=[
            pl.BlockSpec((R, _H), lambda i: (i, 0)),
            pl.BlockSpec((R, _H), lambda i: (i, 0)),
            pl.BlockSpec((R, 2), lambda i: (i, 0)),
            full((2 * _H + 2, _H)),
            full((2 * _H + 2, _H)),
            full((1, _H)),
            full((1, _H)),
        ],
        out_specs=pl.BlockSpec((R, _H), lambda i: (i, 0)),
        out_shape=jax.ShapeDtypeStruct((_EP, _H), jnp.float32),
    )(u, v, ea, wf, ws, bf, bs)


def _sum_body(a0_ref, a1_ref, s_ref):
    i = pl.program_id(0)

    @pl.when(i == 0)
    def _():
        s_ref[:] = jnp.zeros_like(s_ref)

    a = a0_ref[:] + a1_ref[:]
    s_ref[:] += jnp.sum(a, axis=0, keepdims=True)


def _var_body(a0_ref, a1_ref, s_ref, v_ref):
    i = pl.program_id(0)

    @pl.when(i == 0)
    def _():
        v_ref[:] = jnp.zeros_like(v_ref)

    d = a0_ref[:] + a1_ref[:] - s_ref[:] * (1.0 / _N)
    v_ref[:] += jnp.sum(d * d, axis=0, keepdims=True)


def _stats(a0, a1):
    R = 1000
    row = pl.BlockSpec((R, _H), lambda i: (i, 0))
    one = pl.BlockSpec((1, _H), lambda i: (0, 0))
    s = pl.pallas_call(
        _sum_body,
        grid=(_N // R,),
        in_specs=[row, row],
        out_specs=one,
        out_shape=jax.ShapeDtypeStruct((1, _H), jnp.float32),
    )(a0, a1)
    v = pl.pallas_call(
        _var_body,
        grid=(_N // R,),
        in_specs=[row, row, one],
        out_specs=one,
        out_shape=jax.ShapeDtypeStruct((1, _H), jnp.float32),
    )(a0, a1, s)
    return s, v


def _norm_scale(s, ss, g_ref):
    mu = s * (1.0 / _N)
    var = ss * (1.0 / _N)
    return mu, g_ref[:] * lax.rsqrt(var + 1e-5)


def _bn_body(a0_ref, a1_ref, xn_ref, s_ref, ss_ref, g_ref, b_ref, f_ref):
    mu, scale = _norm_scale(s_ref[:], ss_ref[:], g_ref)
    f_ref[:] = xn_ref[:] + (a0_ref[:] + a1_ref[:] - mu) * scale + b_ref[:]


def _bn(a0, a1, xn, gamma, beta):
    s, ss = _stats(a0, a1)
    R = 2000
    full = lambda s_: pl.BlockSpec(s_, lambda i: (0, 0))
    row = lambda w: pl.BlockSpec((R, w), lambda i: (i, 0))
    return pl.pallas_call(
        _bn_body,
        grid=(_N // R,),
        in_specs=[
            row(_H), row(_H), row(_H), full((1, _H)), full((1, _H)),
            full((1, _H)), full((1, _H)),
        ],
        out_specs=row(_H),
        out_shape=jax.ShapeDtypeStruct((_N, _H), jnp.float32),
    )(a0, a1, xn, s, ss, gamma, beta)


def _bn_tgt_body(a0_ref, a1_ref, xn_ref, s_ref, ss_ref, g_ref, b_ref,
                 batch_ref, f_ref, tgt_ref):
    i = pl.program_id(0)
    mu, scale = _norm_scale(s_ref[:], ss_ref[:], g_ref)
    f_ref[:] = (xn_ref[:] + (a0_ref[:] + a1_ref[:] - mu) * scale + b_ref[:])

    @pl.when(i == 0)
    def _():
        tgt_ref[:] = jnp.zeros_like(tgt_ref)

    gids = lax.broadcasted_iota(jnp.int32, (1, _G), 1)
    b = batch_ref[:]
    tgt_ref[:] += jnp.sum((b < gids).astype(jnp.int32), axis=0, keepdims=True)

    @pl.when(i == pl.num_programs(0) - 1)
    def _():
        tgt_ref[:] = jnp.minimum(tgt_ref[:], _N - 1)


def _bn_tgt(a0, a1, xn, gamma, beta, batch_col):
    s, ss = _stats(a0, a1)
    R = 2000
    full = lambda s_: pl.BlockSpec(s_, lambda i: (0, 0))
    row = lambda w: pl.BlockSpec((R, w), lambda i: (i, 0))
    return pl.pallas_call(
        _bn_tgt_body,
        grid=(_N // R,),
        in_specs=[
            row(_H), row(_H), row(_H), full((1, _H)), full((1, _H)),
            full((1, _H)), full((1, _H)), row(1),
        ],
        out_specs=[row(_H), pl.BlockSpec((1, _G), lambda i: (0, 0))],
        out_shape=[
            jax.ShapeDtypeStruct((_N, _H), jnp.float32),
            jax.ShapeDtypeStruct((1, _G), jnp.int32),
        ],
    )(a0, a1, xn, s, ss, gamma, beta, batch_col)


def _decode_body(hist_ref, f2_ref, tgt_ref, wnbr_ref, bnbr_ref, w1ih_ref,
                 w1hh_ref, b1_ref, w2ih_ref, w2hh_ref, b2_ref, wop_ref,
                 bop_ref, out_ref, h2all):
    ht = jnp.zeros((_G, _H), jnp.float32)
    ft = jnp.zeros((_G, _H), jnp.float32)
    C = 1000
    dn = (((0,), (0,)), ((), ()))
    for ci in range(_N // C):
        ni = lax.broadcasted_iota(jnp.int32, (C, 1), 0) + ci * C
        oh = (ni == tgt_ref[:]).astype(jnp.float32)
        ht += lax.dot_general(oh, hist_ref[pl.ds(ci * C, C), :], dn,
                              precision=_PREC,
                              preferred_element_type=jnp.float32)
        ft += lax.dot_general(oh, f2_ref[pl.ds(ci * C, C), :], dn,
                              precision=_PREC,
                              preferred_element_type=jnp.float32)

    tar = _leaky(_dotd(ft, wnbr_ref[:]) + bnbr_ref[:])
    enc = jnp.concatenate([ht, tar], axis=1)
    gi1 = _dotd(enc, w1ih_ref[:]) + b1_ref[:]
    h1 = jnp.zeros((_G, _D), jnp.float32)
    c1 = jnp.zeros((_G, _D), jnp.float32)
    h2 = jnp.zeros((_G, _D), jnp.float32)
    c2 = jnp.zeros((_G, _D), jnp.float32)
    for t in range(_OUT):
        g1 = gi1 + _dotd(h1, w1hh_ref[:])
        i1 = jax.nn.sigmoid(g1[:, :_D])
        f1 = jax.nn.sigmoid(g1[:, _D:2 * _D])
        gg1 = jnp.tanh(g1[:, 2 * _D:3 * _D])
        o1 = jax.nn.sigmoid(g1[:, 3 * _D:])
        c1 = f1 * c1 + i1 * gg1
        h1 = o1 * jnp.tanh(c1)
        g2 = _dotd(h1, w2ih_ref[:]) + _dotd(h2, w2hh_ref[:]) + b2_ref[:]
        i2 = jax.nn.sigmoid(g2[:, :_D])
        f2g = jax.nn.sigmoid(g2[:, _D:2 * _D])
        gg2 = jnp.tanh(g2[:, 2 * _D:3 * _D])
        o2 = jax.nn.sigmoid(g2[:, 3 * _D:])
        c2 = f2g * c2 + i2 * gg2
        h2 = o2 * jnp.tanh(c2)
        h2all[:, t * _D:(t + 1) * _D] = h2
    out_ref[:] = _dotd(h2all[:], wop_ref[:]) + bop_ref[:]


def _decode(hist, f2, tgt, wnbr, bnbr, w1ih, w1hh, b1, w2ih, w2hh, b2, wop,
            bop):
    full = lambda s: pl.BlockSpec(s, lambda: (0, 0))
    return pl.pallas_call(
        _decode_body,
        in_specs=[
            full((_N, _H)), full((_N, _H)), full((1, _G)),
            full((_H, _H)), full((1, _H)),
            full((2 * _H, 4 * _D)), full((_D, 4 * _D)), full((1, 4 * _D)),
            full((_D, 4 * _D)), full((_D, 4 * _D)), full((1, 4 * _D)),
            full((_OUT * _D, _OUT * 2)), full((1, _OUT * 2)),
        ],
        out_specs=full((_G, _OUT * 2)),
        out_shape=jax.ShapeDtypeStruct((_G, _OUT * 2), jnp.float32),
        scratch_shapes=[pltpu.VMEM((_G, _OUT * _D), jnp.float32)],
    )(hist, f2, tgt, wnbr, bnbr, w1ih, w1hh, b1, w2ih, w2hh, b2, wop, bop)


def _cgconv_layer(xn, dst2d, src2d, ea_pad, zrows, Wf, bf, Ws, bs):
    xn_pad = jnp.concatenate(
        [xn, jnp.zeros((_NP - _N, _H), jnp.float32)], axis=0)
    u, v = _sc_gather_uv(xn_pad, dst2d, src2d)
    m = _edge_m(u, v, ea_pad, Wf.T, Ws.T, bf[None, :], bs[None, :])
    slabs = _sc_scatter(m, dst2d, zrows)
    return slabs[0], slabs[1]


def kernel(x, edge_index, edge_attr, batch, num_graphs, W_ip, b_ip, gru_Wih,
           gru_Whh, gru_bih, gru_bhh, W_dyn, b_dyn, cgc1_Wf, cgc1_bf, cgc1_Ws,
           cgc1_bs, cgc1_gamma, cgc1_beta, cgc2_Wf, cgc2_bf, cgc2_Ws, cgc2_bs,
           cgc2_gamma, cgc2_beta, W_nbr, b_nbr, lstm1_Wih, lstm1_Whh,
           lstm1_bih, lstm1_bhh, lstm2_Wih, lstm2_Whh, lstm2_bih, lstm2_bhh,
           W_op, b_op):
    emb2 = _embed(x.reshape(_N * _T, 2), W_ip.T, b_ip[None, :])
    emb = emb2.reshape(_N, _T * _EMB)
    hist = _encode(emb, gru_Wih.T, gru_Whh.T, gru_bih[None, :],
                   gru_bhh[None, :], W_dyn.T, b_dyn[None, :])

    pad_e = _EP - _E
    dst2d = jnp.concatenate(
        [edge_index[1], jnp.full((pad_e,), _N, jnp.int32)]).reshape(
            _NW * _NCH, _CH)
    src2d = jnp.concatenate(
        [edge_index[0], jnp.full((pad_e,), _N, jnp.int32)]).reshape(
            _NW * _NCH, _CH)
    ea_pad = jnp.concatenate(
        [edge_attr, jnp.zeros((pad_e, 2), jnp.float32)], axis=0)
    zrows = jnp.zeros((_NP, _H), jnp.float32)

    a0, a1 = _cgconv_layer(hist, dst2d, src2d, ea_pad, zrows,
                           cgc1_Wf, cgc1_bf, cgc1_Ws, cgc1_bs)
    f1 = _bn(a0, a1, hist, cgc1_gamma[None, :], cgc1_beta[None, :])

    b0, b1 = _cgconv_layer(f1, dst2d, src2d, ea_pad, zrows,
                           cgc2_Wf, cgc2_bf, cgc2_Ws, cgc2_bs)
    f2, tgt2d = _bn_tgt(b0, b1, f1, cgc2_gamma[None, :], cgc2_beta[None, :],
                        batch.reshape(_N, 1))

    wop_blk = jax.scipy.linalg.block_diag(*([W_op.T] * _OUT))
    bop = jnp.tile(b_op, _OUT)[None, :]
    b1c = (lstm1_bih + lstm1_bhh)[None, :]
    b2c = (lstm2_bih + lstm2_bhh)[None, :]
    out = _decode(hist, f2, tgt2d, W_nbr.T, b_nbr[None, :], lstm1_Wih.T,
                  lstm1_Whh.T, b1c, lstm2_Wih.T, lstm2_Whh.T, b2c,
                  wop_blk, bop)
    return out.reshape(_G, _OUT, 2)


# speedup vs baseline: 2.1047x; 1.0329x over previous
import functools

import jax
import jax.numpy as jnp
from jax import lax
from jax.experimental import pallas as pl
from jax.experimental.pallas import tpu as pltpu
from jax.experimental.pallas import tpu_sc as plsc

_N = 10000
_E = 320000
_T = 16
_G = 512
_EMB = 32
_H = 128
_D = 128
_OUT = 25

_NW = 32
_CH = 128
_EW = 10240
_EP = _NW * _EW
_NCH = _EW // _CH
_NP = 10240

_PREC = lax.Precision.HIGHEST


def _leaky(v):
    return jnp.where(v > 0, v, 0.1 * v)


def _dot(a, b):
    return jnp.dot(a, b, precision=_PREC, preferred_element_type=jnp.float32)


def _dotd(a, b):
    return jnp.dot(a, b, preferred_element_type=jnp.float32)


def _sc_mesh():
    return plsc.VectorSubcoreMesh(core_axis_name="c", subcore_axis_name="s",
                                  num_cores=2, num_subcores=16)


def _sc_gather_uv(table_pad, dst2d, src2d):
    @functools.partial(
        pl.kernel,
        out_type=[jax.ShapeDtypeStruct((_EP, _H), jnp.float32),
                  jax.ShapeDtypeStruct((_EP, _H), jnp.float32)],
        mesh=_sc_mesh(),
        scratch_types=[
            pltpu.VMEM((_NCH, _CH), jnp.int32),
            pltpu.VMEM((_NCH, _CH), jnp.int32),
            pltpu.VMEM((_CH, _H), jnp.float32),
            pltpu.VMEM((_CH, _H), jnp.float32),
            pltpu.VMEM((_CH, _H), jnp.float32),
            pltpu.VMEM((_CH, _H), jnp.float32),
            pltpu.SemaphoreType.DMA,
            pltpu.SemaphoreType.DMA,
            pltpu.SemaphoreType.DMA,
            pltpu.SemaphoreType.DMA,
            pltpu.SemaphoreType.DMA,
            pltpu.SemaphoreType.DMA,
            pltpu.SemaphoreType.DMA,
            pltpu.SemaphoreType.DMA,
        ],
    )
    def k(tab_hbm, dst_hbm, src_hbm, u_hbm, v_hbm, idxd, idxs, u0, v0, u1,
          v1, su0, sv0, su1, sv1, wu0, wv0, wu1, wv1):
        wid = lax.axis_index("s") * 2 + lax.axis_index("c")
        base0 = pl.multiple_of(wid * _EW, _CH)
        pltpu.sync_copy(dst_hbm.at[pl.ds(wid * _NCH, _NCH)], idxd)
        pltpu.sync_copy(src_hbm.at[pl.ds(wid * _NCH, _NCH)], idxs)
        sets = ((u0, v0, su0, sv0, wu0, wv0),
                (u1, v1, su1, sv1, wu1, wv1))

        def startg(ci, st):
            ub, vb, su, sv, wu, wv = st
            pltpu.async_copy(tab_hbm.at[idxd.at[ci]], ub, su)
            pltpu.async_copy(tab_hbm.at[idxs.at[ci]], vb, sv)

        def finishg(ci, st):
            ub, vb, su, sv, wu, wv = st
            pltpu.make_async_copy(tab_hbm.at[idxd.at[ci]], ub, su).wait()
            pltpu.make_async_copy(tab_hbm.at[idxs.at[ci]], vb, sv).wait()
            base = pl.multiple_of(base0 + ci * _CH, _CH)
            pltpu.async_copy(ub, u_hbm.at[pl.ds(base, _CH)], wu)
            pltpu.async_copy(vb, v_hbm.at[pl.ds(base, _CH)], wv)

        def waitw(st):
            ub, vb, su, sv, wu, wv = st
            pltpu.make_async_copy(ub, u_hbm.at[pl.ds(0, _CH)], wu).wait()
            pltpu.make_async_copy(vb, v_hbm.at[pl.ds(0, _CH)], wv).wait()

        npair = _NCH // 2
        startg(0, sets[0])

        def body(pi, carry):
            c0 = pi * 2

            @pl.when(pi > 0)
            def _():
                waitw(sets[1])

            startg(c0 + 1, sets[1])
            finishg(c0, sets[0])

            @pl.when(pi < npair - 1)
            def _():
                waitw(sets[0])
                startg(c0 + 2, sets[0])

            finishg(c0 + 1, sets[1])
            return carry

        lax.fori_loop(0, npair, body, 0)
        waitw(sets[0])
        waitw(sets[1])

    return k(table_pad, dst2d, src2d)


def _sc_scatter(m, dst_pad, zrows):
    @functools.partial(
        pl.kernel,
        out_type=jax.ShapeDtypeStruct((2, _NP, _H), jnp.float32),
        mesh=_sc_mesh(),
        scratch_types=[
            pltpu.VMEM((_NCH, _CH), jnp.int32),
            pltpu.VMEM((_CH, _H), jnp.float32),
            pltpu.VMEM((_CH, _H), jnp.float32),
            pltpu.VMEM_SHARED((_NP, _H), jnp.float32),
            pltpu.SemaphoreType.DMA,
            pltpu.SemaphoreType.DMA,
        ],
    )
    def k(m_hbm, dst_hbm, z_hbm, out_hbm, idx, m0, m1, acc, sm0, sm1):
        c = lax.axis_index("c")
        s = lax.axis_index("s")
        wid = s * 2 + c
        zr = _NP // 16
        pltpu.sync_copy(z_hbm.at[pl.ds(s * zr, zr)], acc.at[pl.ds(s * zr, zr)])
        pltpu.sync_copy(dst_hbm.at[pl.ds(wid * _NCH, _NCH)], idx)
        plsc.subcore_barrier()

        base0 = pl.multiple_of(wid * _EW, _CH)
        sets = ((m0, sm0), (m1, sm1))

        def start(ci, st):
            mb, sm = st
            base = pl.multiple_of(base0 + ci * _CH, _CH)
            pltpu.async_copy(m_hbm.at[pl.ds(base, _CH)], mb, sm)

        def finish(ci, st):
            mb, sm = st
            base = pl.multiple_of(base0 + ci * _CH, _CH)
            pltpu.make_async_copy(m_hbm.at[pl.ds(base, _CH)], mb, sm).wait()
            pltpu.sync_copy(mb, acc.at[idx.at[ci]], add=True)

        npair = _NCH // 2
        start(0, sets[0])

        def body(pi, carry):
            c0 = pi * 2
            start(c0 + 1, sets[1])
            finish(c0, sets[0])

            @pl.when(pi < npair - 1)
            def _():
                start(c0 + 2, sets[0])

            finish(c0 + 1, sets[1])
            return carry

        lax.fori_loop(0, npair, body, 0)
        plsc.subcore_barrier()
        pltpu.sync_copy(acc.at[pl.ds(s * zr, zr)],
                        out_hbm.at[c, pl.ds(s * zr, zr)])

    return k(m, dst_pad, zrows)


def _embed_body(x2_ref, wip_ref, bip_ref, e_ref):
    e_ref[:] = _leaky(_dotd(x2_ref[:], wip_ref[:]) + bip_ref[:])


def _embed(x2, wip, bip):
    R = 16000
    NT = _N * _T
    return pl.pallas_call(
        _embed_body,
        grid=(NT // R,),
        in_specs=[
            pl.BlockSpec((R, 2), lambda i: (i, 0)),
            pl.BlockSpec((2, _EMB), lambda i: (0, 0)),
            pl.BlockSpec((1, _EMB), lambda i: (0, 0)),
        ],
        out_specs=pl.BlockSpec((R, _EMB), lambda i: (i, 0)),
        out_shape=jax.ShapeDtypeStruct((NT, _EMB), jnp.float32),
    )(x2, wip, bip)


def _encode_body(emb_ref, wih_ref, whh_ref, bih_ref,
                 bhh_ref, wdyn_ref, bdyn_ref, hist_ref):
    R = emb_ref.shape[0]
    h = jnp.zeros((R, _H), jnp.float32)
    for t in range(_T):
        e_t = emb_ref[:, t * _EMB:(t + 1) * _EMB]
        gi = _dotd(e_t, wih_ref[:]) + bih_ref[:]
        gh = _dotd(h, whh_ref[:]) + bhh_ref[:]
        r = jax.nn.sigmoid(gi[:, :_H] + gh[:, :_H])
        z = jax.nn.sigmoid(gi[:, _H:2 * _H] + gh[:, _H:2 * _H])
        n = jnp.tanh(gi[:, 2 * _H:] + r * gh[:, 2 * _H:])
        h = (1.0 - z) * n + z * h
    hist_ref[:] = _leaky(_dotd(_leaky(h), wdyn_ref[:]) + bdyn_ref[:])


def _encode(emb, wih, whh, bih, bhh, wdyn, bdyn):
    R = 2000
    full = lambda s: pl.BlockSpec(s, lambda i: (0, 0))
    return pl.pallas_call(
        _encode_body,
        grid=(_N // R,),
        in_specs=[
            pl.BlockSpec((R, _T * _EMB), lambda i: (i, 0)),
            full((_EMB, 3 * _H)),
            full((_H, 3 * _H)),
            full((1, 3 * _H)),
            full((1, 3 * _H)),
            full((_H, _H)),
            full((1, _H)),
        ],
        out_specs=pl.BlockSpec((R, _H), lambda i: (i, 0)),
        out_shape=jax.ShapeDtypeStruct((_N, _H), jnp.float32),
    )(emb, wih, whh, bih, bhh, wdyn, bdyn)


def _edge_body(u_ref, v_ref, ea_ref, wf_ref, ws_ref, bf_ref, bs_ref, m_ref):
    z = jnp.concatenate([u_ref[:].astype(jnp.float32),
                         v_ref[:].astype(jnp.float32), ea_ref[:]], axis=1)
    gf = _dotd(z, wf_ref[:]) + bf_ref[:]
    gs = _dotd(z, ws_ref[:]) + bs_ref[:]
    sp = jnp.maximum(gs, 0.0) + jnp.log1p(jnp.exp(-jnp.abs(gs)))
    m_ref[:] = jax.nn.sigmoid(gf) * sp


def _edge_m(u, v, ea, wf, ws, bf, bs):
    R = 4096
    grid = _EP // R
    full = lambda s: pl.BlockSpec(s, lambda i: (0, 0))
    return pl.pallas_call(
        _edge_body,
        grid=(grid,),
        in_specs=[
            pl.BlockSpec((R, _H), lambda i: (i, 0)),
            pl.BlockSpec((R, _H), lambda i: (i, 0)),
            pl.BlockSpec((R, 2), lambda i: (i, 0)),
            full((2 * _H + 2, _H)),
            full((2 * _H + 2, _H)),
            full((1, _H)),
            full((1, _H)),
        ],
        out_specs=pl.BlockSpec((R, _H), lambda i: (i, 0)),
        out_shape=jax.ShapeDtypeStruct((_EP, _H), jnp.float32),
    )(u, v, ea, wf, ws, bf, bs)


def _sum_body(a0_ref, a1_ref, s_ref):
    i = pl.program_id(0)

    @pl.when(i == 0)
    def _():
        s_ref[:] = jnp.zeros_like(s_ref)

    a = a0_ref[:] + a1_ref[:]
    s_ref[:] += jnp.sum(a, axis=0, keepdims=True)


def _var_body(a0_ref, a1_ref, s_ref, v_ref):
    i = pl.program_id(0)

    @pl.when(i == 0)
    def _():
        v_ref[:] = jnp.zeros_like(v_ref)

    d = a0_ref[:] + a1_ref[:] - s_ref[:] * (1.0 / _N)
    v_ref[:] += jnp.sum(d * d, axis=0, keepdims=True)


def _stats(a0, a1):
    R = 1000
    row = pl.BlockSpec((R, _H), lambda i: (i, 0))
    one = pl.BlockSpec((1, _H), lambda i: (0, 0))
    s = pl.pallas_call(
        _sum_body,
        grid=(_N // R,),
        in_specs=[row, row],
        out_specs=one,
        out_shape=jax.ShapeDtypeStruct((1, _H), jnp.float32),
    )(a0, a1)
    v = pl.pallas_call(
        _var_body,
        grid=(_N // R,),
        in_specs=[row, row, one],
        out_specs=one,
        out_shape=jax.ShapeDtypeStruct((1, _H), jnp.float32),
    )(a0, a1, s)
    return s, v


def _norm_scale(s, ss, g_ref):
    mu = s * (1.0 / _N)
    var = ss * (1.0 / _N)
    return mu, g_ref[:] * lax.rsqrt(var + 1e-5)


def _bn_body(a0_ref, a1_ref, xn_ref, s_ref, ss_ref, g_ref, b_ref, f_ref):
    mu, scale = _norm_scale(s_ref[:], ss_ref[:], g_ref)
    f_ref[:] = xn_ref[:] + (a0_ref[:] + a1_ref[:] - mu) * scale + b_ref[:]


def _bn(a0, a1, xn, gamma, beta):
    s, ss = _stats(a0, a1)
    R = 2000
    full = lambda s_: pl.BlockSpec(s_, lambda i: (0, 0))
    row = lambda w: pl.BlockSpec((R, w), lambda i: (i, 0))
    return pl.pallas_call(
        _bn_body,
        grid=(_N // R,),
        in_specs=[
            row(_H), row(_H), row(_H), full((1, _H)), full((1, _H)),
            full((1, _H)), full((1, _H)),
        ],
        out_specs=row(_H),
        out_shape=jax.ShapeDtypeStruct((_N, _H), jnp.float32),
    )(a0, a1, xn, s, ss, gamma, beta)


def _bn_tgt_body(a0_ref, a1_ref, xn_ref, s_ref, ss_ref, g_ref, b_ref,
                 batch_ref, f_ref, tgt_ref):
    i = pl.program_id(0)
    mu, scale = _norm_scale(s_ref[:], ss_ref[:], g_ref)
    f_ref[:] = (xn_ref[:] + (a0_ref[:] + a1_ref[:] - mu) * scale + b_ref[:])

    @pl.when(i == 0)
    def _():
        tgt_ref[:] = jnp.zeros_like(tgt_ref)

    gids = lax.broadcasted_iota(jnp.int32, (1, _G), 1)
    b = batch_ref[:]
    tgt_ref[:] += jnp.sum((b < gids).astype(jnp.int32), axis=0, keepdims=True)

    @pl.when(i == pl.num_programs(0) - 1)
    def _():
        tgt_ref[:] = jnp.minimum(tgt_ref[:], _N - 1)


def _bn_tgt(a0, a1, xn, gamma, beta, batch_col):
    s, ss = _stats(a0, a1)
    R = 2000
    full = lambda s_: pl.BlockSpec(s_, lambda i: (0, 0))
    row = lambda w: pl.BlockSpec((R, w), lambda i: (i, 0))
    return pl.pallas_call(
        _bn_tgt_body,
        grid=(_N // R,),
        in_specs=[
            row(_H), row(_H), row(_H), full((1, _H)), full((1, _H)),
            full((1, _H)), full((1, _H)), row(1),
        ],
        out_specs=[row(_H), pl.BlockSpec((1, _G), lambda i: (0, 0))],
        out_shape=[
            jax.ShapeDtypeStruct((_N, _H), jnp.float32),
            jax.ShapeDtypeStruct((1, _G), jnp.int32),
        ],
    )(a0, a1, xn, s, ss, gamma, beta, batch_col)


def _decode_body(hist_ref, f2_ref, tgt_ref, wnbr_ref, bnbr_ref, w1ih_ref,
                 w1hh_ref, b1_ref, w2ih_ref, w2hh_ref, b2_ref, wop_ref,
                 bop_ref, out_ref, h2all):
    ht = jnp.zeros((_G, _H), jnp.float32)
    ft = jnp.zeros((_G, _H), jnp.float32)
    C = 1000
    dn = (((0,), (0,)), ((), ()))
    for ci in range(_N // C):
        ni = lax.broadcasted_iota(jnp.int32, (C, 1), 0) + ci * C
        oh = (ni == tgt_ref[:]).astype(jnp.float32)
        ht += lax.dot_general(oh, hist_ref[pl.ds(ci * C, C), :], dn,
                              precision=_PREC,
                              preferred_element_type=jnp.float32)
        ft += lax.dot_general(oh, f2_ref[pl.ds(ci * C, C), :], dn,
                              precision=_PREC,
                              preferred_element_type=jnp.float32)

    tar = _leaky(_dotd(ft, wnbr_ref[:]) + bnbr_ref[:])
    enc = jnp.concatenate([ht, tar], axis=1)
    gi1 = _dotd(enc, w1ih_ref[:]) + b1_ref[:]
    h1 = jnp.zeros((_G, _D), jnp.float32)
    c1 = jnp.zeros((_G, _D), jnp.float32)
    h2 = jnp.zeros((_G, _D), jnp.float32)
    c2 = jnp.zeros((_G, _D), jnp.float32)
    for t in range(_OUT):
        g1 = gi1 + _dotd(h1, w1hh_ref[:])
        i1 = jax.nn.sigmoid(g1[:, :_D])
        f1 = jax.nn.sigmoid(g1[:, _D:2 * _D])
        gg1 = jnp.tanh(g1[:, 2 * _D:3 * _D])
        o1 = jax.nn.sigmoid(g1[:, 3 * _D:])
        c1 = f1 * c1 + i1 * gg1
        h1 = o1 * jnp.tanh(c1)
        g2 = _dotd(h1, w2ih_ref[:]) + _dotd(h2, w2hh_ref[:]) + b2_ref[:]
        i2 = jax.nn.sigmoid(g2[:, :_D])
        f2g = jax.nn.sigmoid(g2[:, _D:2 * _D])
        gg2 = jnp.tanh(g2[:, 2 * _D:3 * _D])
        o2 = jax.nn.sigmoid(g2[:, 3 * _D:])
        c2 = f2g * c2 + i2 * gg2
        h2 = o2 * jnp.tanh(c2)
        h2all[:, t * _D:(t + 1) * _D] = h2
    out_ref[:] = _dotd(h2all[:], wop_ref[:]) + bop_ref[:]


def _decode(hist, f2, tgt, wnbr, bnbr, w1ih, w1hh, b1, w2ih, w2hh, b2, wop,
            bop):
    full = lambda s: pl.BlockSpec(s, lambda: (0, 0))
    return pl.pallas_call(
        _decode_body,
        in_specs=[
            full((_N, _H)), full((_N, _H)), full((1, _G)),
            full((_H, _H)), full((1, _H)),
            full((2 * _H, 4 * _D)), full((_D, 4 * _D)), full((1, 4 * _D)),
            full((_D, 4 * _D)), full((_D, 4 * _D)), full((1, 4 * _D)),
            full((_OUT * _D, _OUT * 2)), full((1, _OUT * 2)),
        ],
        out_specs=full((_G, _OUT * 2)),
        out_shape=jax.ShapeDtypeStruct((_G, _OUT * 2), jnp.float32),
        scratch_shapes=[pltpu.VMEM((_G, _OUT * _D), jnp.float32)],
    )(hist, f2, tgt, wnbr, bnbr, w1ih, w1hh, b1, w2ih, w2hh, b2, wop, bop)


def _cgconv_layer(xn, dst2d, src2d, ea_pad, zrows, Wf, bf, Ws, bs):
    xn_pad = jnp.concatenate(
        [xn, jnp.zeros((_NP - _N, _H), jnp.float32)], axis=0)
    u, v = _sc_gather_uv(xn_pad, dst2d, src2d)
    m = _edge_m(u, v, ea_pad, Wf.T, Ws.T, bf[None, :], bs[None, :])
    slabs = _sc_scatter(m, dst2d, zrows)
    return slabs[0], slabs[1]


def kernel(x, edge_index, edge_attr, batch, num_graphs, W_ip, b_ip, gru_Wih,
           gru_Whh, gru_bih, gru_bhh, W_dyn, b_dyn, cgc1_Wf, cgc1_bf, cgc1_Ws,
           cgc1_bs, cgc1_gamma, cgc1_beta, cgc2_Wf, cgc2_bf, cgc2_Ws, cgc2_bs,
           cgc2_gamma, cgc2_beta, W_nbr, b_nbr, lstm1_Wih, lstm1_Whh,
           lstm1_bih, lstm1_bhh, lstm2_Wih, lstm2_Whh, lstm2_bih, lstm2_bhh,
           W_op, b_op):
    emb2 = _embed(x.reshape(_N * _T, 2), W_ip.T, b_ip[None, :])
    emb = emb2.reshape(_N, _T * _EMB)
    hist = _encode(emb, gru_Wih.T, gru_Whh.T, gru_bih[None, :],
                   gru_bhh[None, :], W_dyn.T, b_dyn[None, :])

    pad_e = _EP - _E
    dst2d = jnp.concatenate(
        [edge_index[1], jnp.full((pad_e,), _N, jnp.int32)]).reshape(
            _NW * _NCH, _CH)
    src2d = jnp.concatenate(
        [edge_index[0], jnp.full((pad_e,), _N, jnp.int32)]).reshape(
            _NW * _NCH, _CH)
    ea_pad = jnp.concatenate(
        [edge_attr, jnp.zeros((pad_e, 2), jnp.float32)], axis=0)
    zrows = jnp.zeros((_NP, _H), jnp.float32)

    a0, a1 = _cgconv_layer(hist, dst2d, src2d, ea_pad, zrows,
                           cgc1_Wf, cgc1_bf, cgc1_Ws, cgc1_bs)
    f1 = _bn(a0, a1, hist, cgc1_gamma[None, :], cgc1_beta[None, :])

    b0, b1 = _cgconv_layer(f1, dst2d, src2d, ea_pad, zrows,
                           cgc2_Wf, cgc2_bf, cgc2_Ws, cgc2_bs)
    f2, tgt2d = _bn_tgt(b0, b1, f1, cgc2_gamma[None, :], cgc2_beta[None, :],
                        batch.reshape(_N, 1))

    wop_blk = jax.scipy.linalg.block_diag(*([W_op.T] * _OUT))
    bop = jnp.tile(b_op, _OUT)[None, :]
    b1c = (lstm1_bih + lstm1_bhh)[None, :]
    b2c = (lstm2_bih + lstm2_bhh)[None, :]
    out = _decode(hist, f2, tgt2d, W_nbr.T, b_nbr[None, :], lstm1_Wih.T,
                  lstm1_Whh.T, b1c, lstm2_Wih.T, lstm2_Whh.T, b2c,
                  wop_blk, bop)
    return out.reshape(_G, _OUT, 2)
